# baseline (speedup 1.0000x reference)
import functools

import jax
import jax.numpy as jnp
from jax import lax
from jax.experimental import pallas as pl
from jax.experimental.pallas import tpu as pltpu
from jax.experimental.pallas import tpu_sc as plsc

N = 10000
E = 160000
D = 256
DH = 128

NC = 2
NS = 16
NW = NC * NS

_KC = 5
_EC = E // _KC


@functools.cache
def _mesh():
    return plsc.VectorSubcoreMesh(
        core_axis_name="c", subcore_axis_name="s", num_cores=NC, num_subcores=NS)


def _swish(v):
    return v * jax.nn.sigmoid(v)


_GC = 128
_PER_W = _EC // NW
_GN = _PER_W // _GC
_GT = _PER_W - _GN * _GC


def _gather_pair(yi, yj, dst_c, src_c):
    @functools.partial(
        pl.kernel,
        out_type=[jax.ShapeDtypeStruct((_EC, D), jnp.float32)] * 2,
        mesh=_mesh(),
        scratch_types=[
            pltpu.VMEM((_GC,), jnp.int32),
            pltpu.VMEM((_GC,), jnp.int32),
            pltpu.VMEM((_GC, D), jnp.float32),
            pltpu.VMEM((_GC, D), jnp.float32),
            pltpu.VMEM((_GT,), jnp.int32),
            pltpu.VMEM((_GT,), jnp.int32),
            pltpu.VMEM((_GT, D), jnp.float32),
            pltpu.VMEM((_GT, D), jnp.float32),
            pltpu.SemaphoreType.DMA,
        ],
    )
    def k(yi_h, yj_h, dst_h, src_h, oi_h, oj_h,
          id_v, is_v, bi_v, bj_v, id_t, is_t, bi_t, bj_t, sem):
        wid = lax.axis_index("s") * NC + lax.axis_index("c")
        w0 = pl.multiple_of(wid * _PER_W, 8)

        def chunk(base, idv, isv, bi, bj, sz):
            pltpu.sync_copy(dst_h.at[pl.ds(base, sz)], idv)
            pltpu.sync_copy(src_h.at[pl.ds(base, sz)], isv)
            pltpu.async_copy(yi_h.at[idv], bi, sem).wait()
            pltpu.async_copy(yj_h.at[isv], bj, sem).wait()
            pltpu.sync_copy(bi, oi_h.at[pl.ds(base, sz)])
            pltpu.sync_copy(bj, oj_h.at[pl.ds(base, sz)])

        @pl.loop(0, _GN)
        def _(t):
            chunk(pl.multiple_of(w0 + t * _GC, 8), id_v, is_v, bi_v, bj_v, _GC)

        chunk(pl.multiple_of(w0 + _GN * _GC, 8), id_t, is_t, bi_t, bj_t, _GT)

    return k(yi, yj, dst_c, src_c)


_SC_CH = 128
_PER_T = _EC // NS
_SN = _PER_T // _SC_CH
_ST = _PER_T - _SN * _SC_CH
_NP = 10240
_RPT = _NP // NS


def _segment_sum(m2s_chunks, dst, zeros_half):
    @functools.partial(
        pl.kernel,
        out_type=jax.ShapeDtypeStruct((NC, _NP, DH), jnp.float32),
        mesh=_mesh(),
        scratch_types=[
            pltpu.VMEM((_SC_CH,), jnp.int32),
            pltpu.VMEM((_SC_CH, DH), jnp.float32),
            pltpu.VMEM((_ST,), jnp.int32),
            pltpu.VMEM((_ST, DH), jnp.float32),
            pltpu.VMEM_SHARED((_NP, DH), jnp.float32),
            pltpu.SemaphoreType.DMA,
        ],
    )
    def k(*refs):
        ms = refs[:_KC]
        dst_h, z_h, agg_h, idx_v, buf_v, idx_t, buf_t, acc_s, sem = refs[_KC:]
        c = lax.axis_index("c")
        tid = lax.axis_index("s")
        pltpu.sync_copy(z_h.at[pl.ds(tid * _RPT, _RPT)],
                        acc_s.at[pl.ds(tid * _RPT, _RPT)])
        plsc.subcore_barrier()

        for kc in range(_KC):
            m_h = ms[kc]
            g0 = pl.multiple_of(kc * _EC + tid * _PER_T, 8)

            def chunk(off, idv, bufv, sz, m_h=m_h, g0=g0):
                pltpu.sync_copy(dst_h.at[pl.ds(g0 + off, sz)], idv)
                pltpu.sync_copy(m_h.at[c, pl.ds(tid * _PER_T + off, sz), :], bufv)
                pltpu.sync_copy(bufv, acc_s.at[idv], add=True)

            @pl.loop(0, _SN)
            def _(t):
                chunk(t * _SC_CH, idx_v, buf_v, _SC_CH)

            chunk(_SN * _SC_CH, idx_t, buf_t, _ST)

        plsc.subcore_barrier()
        pltpu.sync_copy(acc_s.at[pl.ds(tid * _RPT, _RPT)],
                        agg_h.at[c, pl.ds(tid * _RPT, _RPT), :])

    return k(*m2s_chunks, dst, zeros_half)


_BN = 2000
_BE = 640


def _dot(a, b):
    return jnp.dot(a, b, preferred_element_type=jnp.float32)


def _node_proj(x, wxi, wxj):
    def body(x_r, wi_r, wj_r, yi_r, yj_r):
        xb = x_r[...]
        yi_r[...] = _dot(xb, wi_r[...])
        yj_r[...] = _dot(xb, wj_r[...])

    full = lambda s: pl.BlockSpec(s, lambda i: (0, 0))
    return pl.pallas_call(
        body,
        grid=(N // _BN,),
        in_specs=[pl.BlockSpec((_BN, D), lambda i: (i, 0)), full((D, D)), full((D, D))],
        out_specs=[pl.BlockSpec((_BN, D), lambda i: (i, 0))] * 2,
        out_shape=[jax.ShapeDtypeStruct((N, D), jnp.float32)] * 2,
    )(x, wxi, wxj)


def _message(yi_g, yj_g, amf, edge, ea, w4, we, wa1, wm2, wa2):
    def body(yi_r, yj_r, amf_r, edge_r, ea_r, w4_r, we_r, wa1_r, wm2_r, wa2_r, out_r):
        eab = ea_r[...]
        t = (yi_r[...] + yj_r[...]
             + _dot(amf_r[...], w4_r[...]) + _dot(edge_r[...], we_r[...]))
        m1 = _swish(t * _dot(eab, wa1_r[...]))
        m2 = _swish(_dot(m1, wm2_r[...]) * _dot(eab, wa2_r[...]))
        out_r[0] = m2[:, :DH]
        out_r[1] = m2[:, DH:]

    full = lambda s: pl.BlockSpec(s, lambda i: tuple(0 for _ in s))
    return pl.pallas_call(
        body,
        grid=(_EC // _BE,),
        in_specs=[
            pl.BlockSpec((_BE, D), lambda i: (i, 0)),
            pl.BlockSpec((_BE, D), lambda i: (i, 0)),
            pl.BlockSpec((_BE, 4), lambda i: (i, 0)),
            pl.BlockSpec((_BE, D), lambda i: (i, 0)),
            pl.BlockSpec((_BE, 16), lambda i: (i, 0)),
            full((4, D)), full((D, D)), full((16, D)), full((D, D)), full((16, D)),
        ],
        out_specs=pl.BlockSpec((NC, _BE, DH), lambda i: (0, i, 0)),
        out_shape=jax.ShapeDtypeStruct((NC, _EC, DH), jnp.float32),
    )(yi_g, yj_g, amf, edge, ea, w4, we, wa1, wm2, wa2)


def _node_update(x, agg3, na, wu1a, wu1b, wau1, wu2, wau2, we1a, we1b):
    def body(x_r, ag_r, na_r, wu1a_r, wu1b_r, wau1_r, wu2_r, wau2_r,
             we1a_r, we1b_r, xn_r, ai_r, aj_r):
        xb = x_r[...]
        nab = na_r[...]
        agg = jnp.concatenate([ag_r[0], ag_r[1]], axis=-1)
        u = _swish((_dot(xb, wu1a_r[...]) + _dot(agg, wu1b_r[...]))
                   * _dot(nab, wau1_r[...]))
        u = _dot(u, wu2_r[...]) * _dot(nab, wau2_r[...])
        xn = xb + u
        xn_r[...] = xn
        ai_r[...] = _dot(xn, we1a_r[...])
        aj_r[...] = _dot(xn, we1b_r[...])

    full = lambda s: pl.BlockSpec(s, lambda i: tuple(0 for _ in s))
    return pl.pallas_call(
        body,
        grid=(N // _BN,),
        in_specs=[
            pl.BlockSpec((_BN, D), lambda i: (i, 0)),
            pl.BlockSpec((NC, _BN, DH), lambda i: (0, i, 0)),
            pl.BlockSpec((_BN, 16), lambda i: (i, 0)),
            full((D, D)), full((D, D)), full((16, D)),
            full((D, D)), full((16, D)), full((D, D)), full((D, D)),
        ],
        out_specs=[pl.BlockSpec((_BN, D), lambda i: (i, 0))] * 3,
        out_shape=[jax.ShapeDtypeStruct((N, D), jnp.float32)] * 3,
    )(x, agg3, na, wu1a, wu1b, wau1, wu2, wau2, we1a, we1b)


def _edge_update(ai_g, aj_g, edge, ea, g, wae1, wg1a, wg2a, we2, wae2, wg1b, wg2b):
    def body(ai_r, aj_r, edge_r, ea_r, g_r, wae1_r, wg1a_r, wg2a_r, we2_r, wae2_r,
             wg1b_r, wg2b_r, out_r):
        eab = ea_r[...]
        gb = g_r[...]
        wa = _dot(_swish(_dot(gb, wg1a_r[...])), wg2a_r[...])
        e1 = _swish((ai_r[...] + aj_r[...]) * _dot(eab, wae1_r[...]) * wa)
        wb = _dot(_swish(_dot(gb, wg1b_r[...])), wg2b_r[...])
        e2 = _swish(_dot(e1, we2_r[...]) * _dot(eab, wae2_r[...]) * wb)
        out_r[...] = edge_r[...] + e2

    full = lambda s: pl.BlockSpec(s, lambda i: tuple(0 for _ in s))
    return pl.pallas_call(
        body,
        grid=(_EC // _BE,),
        in_specs=[
            pl.BlockSpec((_BE, D), lambda i: (i, 0)),
            pl.BlockSpec((_BE, D), lambda i: (i, 0)),
            pl.BlockSpec((_BE, D), lambda i: (i, 0)),
            pl.BlockSpec((_BE, 16), lambda i: (i, 0)),
            pl.BlockSpec((_BE, 128), lambda i: (i, 0)),
            full((16, D)), full((128, 64)), full((64, D)), full((D, D)),
            full((16, D)), full((128, 64)), full((64, D)),
        ],
        out_specs=pl.BlockSpec((_BE, D), lambda i: (i, 0)),
        out_shape=jax.ShapeDtypeStruct((_EC, D), jnp.float32),
    )(ai_g, aj_g, edge, ea, g, wae1, wg1a, wg2a, we2, wae2, wg1b, wg2b)


def kernel(x, edge, edge_index, edge_attr, node_attr, additional_message_features,
           edge_dist_gauss, W_m1, Wa_m1, W_m2, Wa_m2, W_u1, Wa_u1, W_u2, Wa_u2,
           W_e1, Wa_e1, Wg1a, Wg2a, W_e2, Wa_e2, Wg1b, Wg2b):
    src = edge_index[0]
    dst = edge_index[1]
    zeros_half = jnp.zeros((_NP, DH), dtype=jnp.float32)

    def cs(a, k):
        return lax.slice_in_dim(a, k * _EC, (k + 1) * _EC, axis=0)

    w4 = W_m1[:4]
    yi, yj = _node_proj(x, W_m1[4:4 + D], W_m1[4 + D:4 + 2 * D])
    m2_chunks = []
    for k in range(_KC):
        yig, yjg = _gather_pair(yi, yj, cs(dst, k), cs(src, k))
        m2_chunks.append(_message(
            yig, yjg, cs(additional_message_features, k), cs(edge, k),
            cs(edge_attr, k), w4, W_m1[4 + 2 * D:], Wa_m1, W_m2, Wa_m2))
    agg3 = _segment_sum(m2_chunks, dst, zeros_half)[:, :N, :]

    x_new, ai, aj = _node_update(x, agg3, node_attr, W_u1[:D], W_u1[D:],
                                 Wa_u1, W_u2, Wa_u2, W_e1[:D], W_e1[D:])

    out_chunks = []
    for k in range(_KC):
        aig, ajg = _gather_pair(ai, aj, cs(dst, k), cs(src, k))
        out_chunks.append(_edge_update(
            aig, ajg, cs(edge, k), cs(edge_attr, k), cs(edge_dist_gauss, k),
            Wa_e1, Wg1a, Wg2a, W_e2, Wa_e2, Wg1b, Wg2b))
    edge_new = jnp.concatenate(out_chunks, axis=0)
    return (x_new, edge_new)


# speedup vs baseline: 1.0950x; 1.0950x over previous
import functools

import jax
import jax.numpy as jnp
from jax import lax
from jax.experimental import pallas as pl
from jax.experimental.pallas import tpu as pltpu
from jax.experimental.pallas import tpu_sc as plsc

N = 10000
E = 160000
D = 256
DH = 128

NC = 2
NS = 16
NW = NC * NS


@functools.cache
def _mesh():
    return plsc.VectorSubcoreMesh(
        core_axis_name="c", subcore_axis_name="s", num_cores=NC, num_subcores=NS)


def _swish(v):
    return v * jax.nn.sigmoid(v)


_GC = 128
_PER_W = E // NW
_GN = _PER_W // _GC
_GT = _PER_W - _GN * _GC


def _gather_pair(yi, yj, dst, src):
    @functools.partial(
        pl.kernel,
        out_type=[jax.ShapeDtypeStruct((E, D), jnp.float32)] * 2,
        mesh=_mesh(),
        scratch_types=[
            pltpu.VMEM((_GC,), jnp.int32),
            pltpu.VMEM((_GC,), jnp.int32),
            pltpu.VMEM((_GC, D), jnp.float32),
            pltpu.VMEM((_GC, D), jnp.float32),
            pltpu.VMEM((_GT,), jnp.int32),
            pltpu.VMEM((_GT,), jnp.int32),
            pltpu.VMEM((_GT, D), jnp.float32),
            pltpu.VMEM((_GT, D), jnp.float32),
            pltpu.SemaphoreType.DMA,
        ],
    )
    def k(yi_h, yj_h, dst_h, src_h, oi_h, oj_h,
          id_v, is_v, bi_v, bj_v, id_t, is_t, bi_t, bj_t, sem):
        wid = lax.axis_index("s") * NC + lax.axis_index("c")
        w0 = pl.multiple_of(wid * _PER_W, 8)

        def chunk(base, idv, isv, bi, bj, sz):
            pltpu.sync_copy(dst_h.at[pl.ds(base, sz)], idv)
            pltpu.sync_copy(src_h.at[pl.ds(base, sz)], isv)
            pltpu.async_copy(yi_h.at[idv], bi, sem).wait()
            pltpu.async_copy(yj_h.at[isv], bj, sem).wait()
            pltpu.sync_copy(bi, oi_h.at[pl.ds(base, sz)])
            pltpu.sync_copy(bj, oj_h.at[pl.ds(base, sz)])

        @pl.loop(0, _GN)
        def _(t):
            chunk(pl.multiple_of(w0 + t * _GC, 8), id_v, is_v, bi_v, bj_v, _GC)

        chunk(pl.multiple_of(w0 + _GN * _GC, 8), id_t, is_t, bi_t, bj_t, _GT)

    return k(yi, yj, dst, src)


_SC_CH = 128
_ROWS_A = 80
_ROWS_B = 50
_NP = 10240
_RPT = _NP // NS


def _segment_sum(m2s, dst2, zeros_half):
    @functools.partial(
        pl.kernel,
        out_type=jax.ShapeDtypeStruct((NC, _NP, DH), jnp.float32),
        mesh=_mesh(),
        scratch_types=[
            pltpu.VMEM((_ROWS_A, _SC_CH), jnp.int32),
            pltpu.VMEM((_SC_CH, DH), jnp.float32),
            pltpu.VMEM((_SC_CH, DH), jnp.float32),
            pltpu.VMEM_SHARED((_NP, DH), jnp.float32),
            pltpu.SemaphoreType.DMA,
            pltpu.SemaphoreType.DMA,
            pltpu.SemaphoreType.DMA,
        ],
    )
    def k(m2s_h, dst2_h, z_h, agg_h, idx_a, b0, b1, acc_s, sz, s0, s1):
        c = lax.axis_index("c")
        tid = lax.axis_index("s")
        pltpu.sync_copy(z_h.at[pl.ds(tid * _RPT, _RPT)],
                        acc_s.at[pl.ds(tid * _RPT, _RPT)])

        @pl.when(tid < NS - 1)
        def _():
            pltpu.sync_copy(dst2_h.at[pl.ds(tid * _ROWS_A, _ROWS_A), :], idx_a)

        @pl.when(tid == NS - 1)
        def _():
            pltpu.sync_copy(dst2_h.at[pl.ds((NS - 1) * _ROWS_A, _ROWS_B), :],
                            idx_a.at[pl.ds(0, _ROWS_B), :])

        plsc.subcore_barrier()

        row0 = tid * _ROWS_A

        def src_at(r):
            base = pl.multiple_of(
                lax.min((row0 + r) * _SC_CH, E - _SC_CH), 8)
            return m2s_h.at[c, pl.ds(base, _SC_CH), :]

        def pipeline(npairs):
            pltpu.async_copy(src_at(0), b0, s0)

            @pl.loop(0, npairs)
            def _(p):
                r0 = 2 * p
                r1 = r0 + 1
                pltpu.async_copy(src_at(r1), b1, s1)
                pltpu.make_async_copy(src_at(r0), b0, s0).wait()
                pltpu.sync_copy(b0, acc_s.at[idx_a.at[r0]], add=True)
                pltpu.async_copy(src_at(r0 + 2), b0, s0)
                pltpu.make_async_copy(src_at(r1), b1, s1).wait()
                pltpu.sync_copy(b1, acc_s.at[idx_a.at[r1]], add=True)

            pltpu.make_async_copy(src_at(2 * npairs), b0, s0).wait()

        @pl.when(tid < NS - 1)
        def _():
            pipeline(_ROWS_A // 2)

        @pl.when(tid == NS - 1)
        def _():
            pipeline(_ROWS_B // 2)

        plsc.subcore_barrier()
        pltpu.sync_copy(acc_s.at[pl.ds(tid * _RPT, _RPT)],
                        agg_h.at[c, pl.ds(tid * _RPT, _RPT), :])

    return k(m2s, dst2, zeros_half)


_BN = 2000
_BE = 640


def _dot(a, b):
    return jnp.dot(a, b, preferred_element_type=jnp.float32)


def _node_proj(x, wxi, wxj):
    def body(x_r, wi_r, wj_r, yi_r, yj_r):
        xb = x_r[...]
        yi_r[...] = _dot(xb, wi_r[...])
        yj_r[...] = _dot(xb, wj_r[...])

    full = lambda s: pl.BlockSpec(s, lambda i: (0, 0))
    return pl.pallas_call(
        body,
        grid=(N // _BN,),
        in_specs=[pl.BlockSpec((_BN, D), lambda i: (i, 0)), full((D, D)), full((D, D))],
        out_specs=[pl.BlockSpec((_BN, D), lambda i: (i, 0))] * 2,
        out_shape=[jax.ShapeDtypeStruct((N, D), jnp.float32)] * 2,
    )(x, wxi, wxj)


def _message(yi_g, yj_g, amf, edge, ea, w4, we, wa1, wm2, wa2):
    def body(yi_r, yj_r, amf_r, edge_r, ea_r, w4_r, we_r, wa1_r, wm2_r, wa2_r, out_r):
        eab = ea_r[...]
        t = (yi_r[...] + yj_r[...]
             + _dot(amf_r[...], w4_r[...]) + _dot(edge_r[...], we_r[...]))
        m1 = _swish(t * _dot(eab, wa1_r[...]))
        m2 = _swish(_dot(m1, wm2_r[...]) * _dot(eab, wa2_r[...]))
        out_r[0] = m2[:, :DH]
        out_r[1] = m2[:, DH:]

    full = lambda s: pl.BlockSpec(s, lambda i: tuple(0 for _ in s))
    return pl.pallas_call(
        body,
        grid=(E // _BE,),
        in_specs=[
            pl.BlockSpec((_BE, D), lambda i: (i, 0)),
            pl.BlockSpec((_BE, D), lambda i: (i, 0)),
            pl.BlockSpec((_BE, 4), lambda i: (i, 0)),
            pl.BlockSpec((_BE, D), lambda i: (i, 0)),
            pl.BlockSpec((_BE, 16), lambda i: (i, 0)),
            full((4, D)), full((D, D)), full((16, D)), full((D, D)), full((16, D)),
        ],
        out_specs=pl.BlockSpec((NC, _BE, DH), lambda i: (0, i, 0)),
        out_shape=jax.ShapeDtypeStruct((NC, E, DH), jnp.float32),
    )(yi_g, yj_g, amf, edge, ea, w4, we, wa1, wm2, wa2)


def _node_update(x, agg3, na, wu1a, wu1b, wau1, wu2, wau2, we1a, we1b):
    def body(x_r, ag_r, na_r, wu1a_r, wu1b_r, wau1_r, wu2_r, wau2_r,
             we1a_r, we1b_r, xn_r, ai_r, aj_r):
        xb = x_r[...]
        nab = na_r[...]
        agg = jnp.concatenate([ag_r[0], ag_r[1]], axis=-1)
        u = _swish((_dot(xb, wu1a_r[...]) + _dot(agg, wu1b_r[...]))
                   * _dot(nab, wau1_r[...]))
        u = _dot(u, wu2_r[...]) * _dot(nab, wau2_r[...])
        xn = xb + u
        xn_r[...] = xn
        ai_r[...] = _dot(xn, we1a_r[...])
        aj_r[...] = _dot(xn, we1b_r[...])

    full = lambda s: pl.BlockSpec(s, lambda i: tuple(0 for _ in s))
    return pl.pallas_call(
        body,
        grid=(N // _BN,),
        in_specs=[
            pl.BlockSpec((_BN, D), lambda i: (i, 0)),
            pl.BlockSpec((NC, _BN, DH), lambda i: (0, i, 0)),
            pl.BlockSpec((_BN, 16), lambda i: (i, 0)),
            full((D, D)), full((D, D)), full((16, D)),
            full((D, D)), full((16, D)), full((D, D)), full((D, D)),
        ],
        out_specs=[pl.BlockSpec((_BN, D), lambda i: (i, 0))] * 3,
        out_shape=[jax.ShapeDtypeStruct((N, D), jnp.float32)] * 3,
    )(x, agg3, na, wu1a, wu1b, wau1, wu2, wau2, we1a, we1b)


def _edge_update(ai_g, aj_g, edge, ea, g, wae1, wg1a, wg2a, we2, wae2, wg1b, wg2b):
    def body(ai_r, aj_r, edge_r, ea_r, g_r, wae1_r, wg1a_r, wg2a_r, we2_r, wae2_r,
             wg1b_r, wg2b_r, out_r):
        eab = ea_r[...]
        gb = g_r[...]
        wa = _dot(_swish(_dot(gb, wg1a_r[...])), wg2a_r[...])
        e1 = _swish((ai_r[...] + aj_r[...]) * _dot(eab, wae1_r[...]) * wa)
        wb = _dot(_swish(_dot(gb, wg1b_r[...])), wg2b_r[...])
        e2 = _swish(_dot(e1, we2_r[...]) * _dot(eab, wae2_r[...]) * wb)
        out_r[...] = edge_r[...] + e2

    full = lambda s: pl.BlockSpec(s, lambda i: tuple(0 for _ in s))
    return pl.pallas_call(
        body,
        grid=(E // _BE,),
        in_specs=[
            pl.BlockSpec((_BE, D), lambda i: (i, 0)),
            pl.BlockSpec((_BE, D), lambda i: (i, 0)),
            pl.BlockSpec((_BE, D), lambda i: (i, 0)),
            pl.BlockSpec((_BE, 16), lambda i: (i, 0)),
            pl.BlockSpec((_BE, 128), lambda i: (i, 0)),
            full((16, D)), full((128, 64)), full((64, D)), full((D, D)),
            full((16, D)), full((128, 64)), full((64, D)),
        ],
        out_specs=pl.BlockSpec((_BE, D), lambda i: (i, 0)),
        out_shape=jax.ShapeDtypeStruct((E, D), jnp.float32),
    )(ai_g, aj_g, edge, ea, g, wae1, wg1a, wg2a, we2, wae2, wg1b, wg2b)


def kernel(x, edge, edge_index, edge_attr, node_attr, additional_message_features,
           edge_dist_gauss, W_m1, Wa_m1, W_m2, Wa_m2, W_u1, Wa_u1, W_u2, Wa_u2,
           W_e1, Wa_e1, Wg1a, Wg2a, W_e2, Wa_e2, Wg1b, Wg2b):
    src = edge_index[0]
    dst = edge_index[1]
    dst2 = dst.reshape(E // _SC_CH, _SC_CH)
    zeros_half = jnp.zeros((_NP, DH), dtype=jnp.float32)

    w4 = W_m1[:4]
    yi, yj = _node_proj(x, W_m1[4:4 + D], W_m1[4 + D:4 + 2 * D])
    yig, yjg = _gather_pair(yi, yj, dst, src)
    m2s = _message(yig, yjg, additional_message_features, edge, edge_attr,
                   w4, W_m1[4 + 2 * D:], Wa_m1, W_m2, Wa_m2)
    agg3 = _segment_sum(m2s, dst2, zeros_half)[:, :N, :]

    x_new, ai, aj = _node_update(x, agg3, node_attr, W_u1[:D], W_u1[D:],
                                 Wa_u1, W_u2, Wa_u2, W_e1[:D], W_e1[D:])

    aig, ajg = _gather_pair(ai, aj, dst, src)
    edge_new = _edge_update(aig, ajg, edge, edge_attr, edge_dist_gauss,
                            Wa_e1, Wg1a, Wg2a, W_e2, Wa_e2, Wg1b, Wg2b)
    return (x_new, edge_new)


# speedup vs baseline: 1.1957x; 1.0920x over previous
import functools

import jax
import jax.numpy as jnp
from jax import lax
from jax.experimental import pallas as pl
from jax.experimental.pallas import tpu as pltpu
from jax.experimental.pallas import tpu_sc as plsc

N = 10000
E = 160000
D = 256
DH = 128

NC = 2
NS = 16
NW = NC * NS


@functools.cache
def _mesh():
    return plsc.VectorSubcoreMesh(
        core_axis_name="c", subcore_axis_name="s", num_cores=NC, num_subcores=NS)


def _swish(v):
    return v * jax.nn.sigmoid(v)


_GC = 112
_PER_W = E // NW
_GN = _PER_W // _GC
_GP = _GN // 2
_GT = _PER_W - _GN * _GC


def _gather_pair(yi, yj, dst, src):
    @functools.partial(
        pl.kernel,
        out_type=[jax.ShapeDtypeStruct((E, D), jnp.float32)] * 2,
        mesh=_mesh(),
        scratch_types=[
            pltpu.VMEM((_GC,), jnp.int32),
            pltpu.VMEM((_GC,), jnp.int32),
            pltpu.VMEM((_GC,), jnp.int32),
            pltpu.VMEM((_GC,), jnp.int32),
            pltpu.VMEM((_GC, D), jnp.float32),
            pltpu.VMEM((_GC, D), jnp.float32),
            pltpu.VMEM((_GC, D), jnp.float32),
            pltpu.VMEM((_GC, D), jnp.float32),
            pltpu.VMEM((_GT,), jnp.int32),
            pltpu.VMEM((_GT,), jnp.int32),
            pltpu.SemaphoreType.DMA,
            pltpu.SemaphoreType.DMA,
            pltpu.SemaphoreType.DMA,
            pltpu.SemaphoreType.DMA,
            pltpu.SemaphoreType.DMA,
        ],
    )
    def k(yi_h, yj_h, dst_h, src_h, oi_h, oj_h,
          id0, is0, id1, is1, bi0, bj0, bi1, bj1, id_t, is_t,
          si0, si1, sg, sw0, sw1):
        wid = lax.axis_index("s") * NC + lax.axis_index("c")
        w0 = pl.multiple_of(wid * _PER_W, 8)

        def base(t):
            return pl.multiple_of(w0 + t * _GC, 8)

        def idx_start(b, idv, isv, sem):
            pltpu.async_copy(dst_h.at[pl.ds(b, _GC)], idv, sem)
            pltpu.async_copy(src_h.at[pl.ds(b, _GC)], isv, sem)

        def idx_wait(b, idv, isv, sem):
            pltpu.make_async_copy(dst_h.at[pl.ds(b, _GC)], idv, sem).wait()
            pltpu.make_async_copy(src_h.at[pl.ds(b, _GC)], isv, sem).wait()

        def out_wait(b, bi, bj, sem):
            pltpu.make_async_copy(bi, oi_h.at[pl.ds(b, _GC)], sem).wait()
            pltpu.make_async_copy(bj, oj_h.at[pl.ds(b, _GC)], sem).wait()

        def process(b, idv, isv, bi, bj, wsem):
            d1 = pltpu.async_copy(yi_h.at[idv], bi, sg)
            d2 = pltpu.async_copy(yj_h.at[isv], bj, sg)
            d1.wait()
            d2.wait()
            pltpu.async_copy(bi, oi_h.at[pl.ds(b, _GC)], wsem)
            pltpu.async_copy(bj, oj_h.at[pl.ds(b, _GC)], wsem)

        idx_start(base(0), id0, is0, si0)

        @pl.loop(0, _GP)
        def _(p):
            b0 = base(2 * p)
            b1 = base(2 * p + 1)
            idx_start(b1, id1, is1, si1)
            idx_wait(b0, id0, is0, si0)

            @pl.when(p > 0)
            def _():
                out_wait(b0, bi0, bj0, sw0)

            process(b0, id0, is0, bi0, bj0, sw0)

            @pl.when(p < _GP - 1)
            def _():
                idx_start(base(2 * p + 2), id0, is0, si0)

            idx_wait(b1, id1, is1, si1)

            @pl.when(p > 0)
            def _():
                out_wait(b1, bi1, bj1, sw1)

            process(b1, id1, is1, bi1, bj1, sw1)

        tb = pl.multiple_of(w0 + _GN * _GC, 8)
        out_wait(base(0), bi0, bj0, sw0)
        pltpu.sync_copy(dst_h.at[pl.ds(tb, _GT)], id_t)
        pltpu.sync_copy(src_h.at[pl.ds(tb, _GT)], is_t)
        d1 = pltpu.async_copy(yi_h.at[id_t], bi0.at[pl.ds(0, _GT), :], sg)
        d2 = pltpu.async_copy(yj_h.at[is_t], bj0.at[pl.ds(0, _GT), :], sg)
        d1.wait()
        d2.wait()
        pltpu.sync_copy(bi0.at[pl.ds(0, _GT), :], oi_h.at[pl.ds(tb, _GT)])
        pltpu.sync_copy(bj0.at[pl.ds(0, _GT), :], oj_h.at[pl.ds(tb, _GT)])
        out_wait(base(0), bi1, bj1, sw1)

    return k(yi, yj, dst, src)


_SC_CH = 128
_ROWS_A = 80
_ROWS_B = 50
_NP = 10240
_RPT = _NP // NS


def _segment_sum(m2s, dst2, zeros_half):
    @functools.partial(
        pl.kernel,
        out_type=jax.ShapeDtypeStruct((NC, _NP, DH), jnp.float32),
        mesh=_mesh(),
        scratch_types=[
            pltpu.VMEM((_ROWS_A, _SC_CH), jnp.int32),
            pltpu.VMEM((_SC_CH, DH), jnp.float32),
            pltpu.VMEM((_SC_CH, DH), jnp.float32),
            pltpu.VMEM_SHARED((_NP, DH), jnp.float32),
            pltpu.SemaphoreType.DMA,
            pltpu.SemaphoreType.DMA,
            pltpu.SemaphoreType.DMA,
        ],
    )
    def k(m2s_h, dst2_h, z_h, agg_h, idx_a, b0, b1, acc_s, sz, s0, s1):
        c = lax.axis_index("c")
        tid = lax.axis_index("s")
        pltpu.sync_copy(z_h.at[pl.ds(tid * _RPT, _RPT)],
                        acc_s.at[pl.ds(tid * _RPT, _RPT)])

        @pl.when(tid < NS - 1)
        def _():
            pltpu.sync_copy(dst2_h.at[pl.ds(tid * _ROWS_A, _ROWS_A), :], idx_a)

        @pl.when(tid == NS - 1)
        def _():
            pltpu.sync_copy(dst2_h.at[pl.ds((NS - 1) * _ROWS_A, _ROWS_B), :],
                            idx_a.at[pl.ds(0, _ROWS_B), :])

        plsc.subcore_barrier()

        row0 = tid * _ROWS_A

        def src_at(r):
            base = pl.multiple_of(
                lax.min((row0 + r) * _SC_CH, E - _SC_CH), 8)
            return m2s_h.at[c, pl.ds(base, _SC_CH), :]

        def pipeline(npairs):
            pltpu.async_copy(src_at(0), b0, s0)

            @pl.loop(0, npairs)
            def _(p):
                r0 = 2 * p
                r1 = r0 + 1
                pltpu.async_copy(src_at(r1), b1, s1)
                pltpu.make_async_copy(src_at(r0), b0, s0).wait()
                pltpu.sync_copy(b0, acc_s.at[idx_a.at[r0]], add=True)
                pltpu.async_copy(src_at(r0 + 2), b0, s0)
                pltpu.make_async_copy(src_at(r1), b1, s1).wait()
                pltpu.sync_copy(b1, acc_s.at[idx_a.at[r1]], add=True)

            pltpu.make_async_copy(src_at(2 * npairs), b0, s0).wait()

        @pl.when(tid < NS - 1)
        def _():
            pipeline(_ROWS_A // 2)

        @pl.when(tid == NS - 1)
        def _():
            pipeline(_ROWS_B // 2)

        plsc.subcore_barrier()
        pltpu.sync_copy(acc_s.at[pl.ds(tid * _RPT, _RPT)],
                        agg_h.at[c, pl.ds(tid * _RPT, _RPT), :])

    return k(m2s, dst2, zeros_half)


_BN = 2000
_BE = 640


def _dot(a, b):
    return jnp.dot(a, b, preferred_element_type=jnp.float32)


def _node_proj(x, wxi, wxj):
    def body(x_r, wi_r, wj_r, yi_r, yj_r):
        xb = x_r[...]
        yi_r[...] = _dot(xb, wi_r[...])
        yj_r[...] = _dot(xb, wj_r[...])

    full = lambda s: pl.BlockSpec(s, lambda i: (0, 0))
    return pl.pallas_call(
        body,
        grid=(N // _BN,),
        in_specs=[pl.BlockSpec((_BN, D), lambda i: (i, 0)), full((D, D)), full((D, D))],
        out_specs=[pl.BlockSpec((_BN, D), lambda i: (i, 0))] * 2,
        out_shape=[jax.ShapeDtypeStruct((N, D), jnp.float32)] * 2,
    )(x, wxi, wxj)


def _message(yi_g, yj_g, amf, edge, ea, w4, we, wa1, wm2, wa2):
    def body(yi_r, yj_r, amf_r, edge_r, ea_r, w4_r, we_r, wa1_r, wm2_r, wa2_r, out_r):
        eab = ea_r[...]
        t = (yi_r[...] + yj_r[...]
             + _dot(amf_r[...], w4_r[...]) + _dot(edge_r[...], we_r[...]))
        m1 = _swish(t * _dot(eab, wa1_r[...]))
        m2 = _swish(_dot(m1, wm2_r[...]) * _dot(eab, wa2_r[...]))
        out_r[0] = m2[:, :DH]
        out_r[1] = m2[:, DH:]

    full = lambda s: pl.BlockSpec(s, lambda i: tuple(0 for _ in s))
    return pl.pallas_call(
        body,
        grid=(E // _BE,),
        in_specs=[
            pl.BlockSpec((_BE, D), lambda i: (i, 0)),
            pl.BlockSpec((_BE, D), lambda i: (i, 0)),
            pl.BlockSpec((_BE, 4), lambda i: (i, 0)),
            pl.BlockSpec((_BE, D), lambda i: (i, 0)),
            pl.BlockSpec((_BE, 16), lambda i: (i, 0)),
            full((4, D)), full((D, D)), full((16, D)), full((D, D)), full((16, D)),
        ],
        out_specs=pl.BlockSpec((NC, _BE, DH), lambda i: (0, i, 0)),
        out_shape=jax.ShapeDtypeStruct((NC, E, DH), jnp.float32),
    )(yi_g, yj_g, amf, edge, ea, w4, we, wa1, wm2, wa2)


def _node_update(x, agg3, na, wu1a, wu1b, wau1, wu2, wau2, we1a, we1b):
    def body(x_r, ag_r, na_r, wu1a_r, wu1b_r, wau1_r, wu2_r, wau2_r,
             we1a_r, we1b_r, xn_r, ai_r, aj_r):
        xb = x_r[...]
        nab = na_r[...]
        agg = jnp.concatenate([ag_r[0], ag_r[1]], axis=-1)
        u = _swish((_dot(xb, wu1a_r[...]) + _dot(agg, wu1b_r[...]))
                   * _dot(nab, wau1_r[...]))
        u = _dot(u, wu2_r[...]) * _dot(nab, wau2_r[...])
        xn = xb + u
        xn_r[...] = xn
        ai_r[...] = _dot(xn, we1a_r[...])
        aj_r[...] = _dot(xn, we1b_r[...])

    full = lambda s: pl.BlockSpec(s, lambda i: tuple(0 for _ in s))
    return pl.pallas_call(
        body,
        grid=(N // _BN,),
        in_specs=[
            pl.BlockSpec((_BN, D), lambda i: (i, 0)),
            pl.BlockSpec((NC, _BN, DH), lambda i: (0, i, 0)),
            pl.BlockSpec((_BN, 16), lambda i: (i, 0)),
            full((D, D)), full((D, D)), full((16, D)),
            full((D, D)), full((16, D)), full((D, D)), full((D, D)),
        ],
        out_specs=[pl.BlockSpec((_BN, D), lambda i: (i, 0))] * 3,
        out_shape=[jax.ShapeDtypeStruct((N, D), jnp.float32)] * 3,
    )(x, agg3, na, wu1a, wu1b, wau1, wu2, wau2, we1a, we1b)


def _edge_update(ai_g, aj_g, edge, ea, g, wae1, wg1a, wg2a, we2, wae2, wg1b, wg2b):
    def body(ai_r, aj_r, edge_r, ea_r, g_r, wae1_r, wg1a_r, wg2a_r, we2_r, wae2_r,
             wg1b_r, wg2b_r, out_r):
        eab = ea_r[...]
        gb = g_r[...]
        wa = _dot(_swish(_dot(gb, wg1a_r[...])), wg2a_r[...])
        e1 = _swish((ai_r[...] + aj_r[...]) * _dot(eab, wae1_r[...]) * wa)
        wb = _dot(_swish(_dot(gb, wg1b_r[...])), wg2b_r[...])
        e2 = _swish(_dot(e1, we2_r[...]) * _dot(eab, wae2_r[...]) * wb)
        out_r[...] = edge_r[...] + e2

    full = lambda s: pl.BlockSpec(s, lambda i: tuple(0 for _ in s))
    return pl.pallas_call(
        body,
        grid=(E // _BE,),
        in_specs=[
            pl.BlockSpec((_BE, D), lambda i: (i, 0)),
            pl.BlockSpec((_BE, D), lambda i: (i, 0)),
            pl.BlockSpec((_BE, D), lambda i: (i, 0)),
            pl.BlockSpec((_BE, 16), lambda i: (i, 0)),
            pl.BlockSpec((_BE, 128), lambda i: (i, 0)),
            full((16, D)), full((128, 64)), full((64, D)), full((D, D)),
            full((16, D)), full((128, 64)), full((64, D)),
        ],
        out_specs=pl.BlockSpec((_BE, D), lambda i: (i, 0)),
        out_shape=jax.ShapeDtypeStruct((E, D), jnp.float32),
    )(ai_g, aj_g, edge, ea, g, wae1, wg1a, wg2a, we2, wae2, wg1b, wg2b)


def kernel(x, edge, edge_index, edge_attr, node_attr, additional_message_features,
           edge_dist_gauss, W_m1, Wa_m1, W_m2, Wa_m2, W_u1, Wa_u1, W_u2, Wa_u2,
           W_e1, Wa_e1, Wg1a, Wg2a, W_e2, Wa_e2, Wg1b, Wg2b):
    src = edge_index[0]
    dst = edge_index[1]
    dst2 = dst.reshape(E // _SC_CH, _SC_CH)
    zeros_half = jnp.zeros((_NP, DH), dtype=jnp.float32)

    w4 = W_m1[:4]
    yi, yj = _node_proj(x, W_m1[4:4 + D], W_m1[4 + D:4 + 2 * D])
    yig, yjg = _gather_pair(yi, yj, dst, src)
    m2s = _message(yig, yjg, additional_message_features, edge, edge_attr,
                   w4, W_m1[4 + 2 * D:], Wa_m1, W_m2, Wa_m2)
    agg3 = _segment_sum(m2s, dst2, zeros_half)[:, :N, :]

    x_new, ai, aj = _node_update(x, agg3, node_attr, W_u1[:D], W_u1[D:],
                                 Wa_u1, W_u2, Wa_u2, W_e1[:D], W_e1[D:])

    aig, ajg = _gather_pair(ai, aj, dst, src)
    edge_new = _edge_update(aig, ajg, edge, edge_attr, edge_dist_gauss,
                            Wa_e1, Wg1a, Wg2a, W_e2, Wa_e2, Wg1b, Wg2b)
    return (x_new, edge_new)


# speedup vs baseline: 1.3585x; 1.1361x over previous
import functools

import jax
import jax.numpy as jnp
from jax import lax
from jax.experimental import pallas as pl
from jax.experimental.pallas import tpu as pltpu
from jax.experimental.pallas import tpu_sc as plsc

N = 10000
E = 160000
D = 256
DH = 128

NC = 2
NS = 16
NW = NC * NS


@functools.cache
def _mesh():
    return plsc.VectorSubcoreMesh(
        core_axis_name="c", subcore_axis_name="s", num_cores=NC, num_subcores=NS)


def _swish(v):
    return v * jax.nn.sigmoid(v)


_GC = 112
_PER_W = E // NW
_GN = _PER_W // _GC
_GP = _GN // 2
_GT = _PER_W - _GN * _GC


def _gather_pair(yi, yj, dst, src):
    @functools.partial(
        pl.kernel,
        out_type=[jax.ShapeDtypeStruct((E, D), jnp.float32)] * 2,
        mesh=_mesh(),
        scratch_types=[
            pltpu.VMEM((_GC,), jnp.int32),
            pltpu.VMEM((_GC,), jnp.int32),
            pltpu.VMEM((_GC,), jnp.int32),
            pltpu.VMEM((_GC,), jnp.int32),
            pltpu.VMEM((_GC, D), jnp.float32),
            pltpu.VMEM((_GC, D), jnp.float32),
            pltpu.VMEM((_GC, D), jnp.float32),
            pltpu.VMEM((_GC, D), jnp.float32),
            pltpu.VMEM((_GT,), jnp.int32),
            pltpu.VMEM((_GT,), jnp.int32),
            pltpu.SemaphoreType.DMA,
            pltpu.SemaphoreType.DMA,
            pltpu.SemaphoreType.DMA,
            pltpu.SemaphoreType.DMA,
            pltpu.SemaphoreType.DMA,
        ],
    )
    def k(yi_h, yj_h, dst_h, src_h, oi_h, oj_h,
          id0, is0, id1, is1, bi0, bj0, bi1, bj1, id_t, is_t,
          si0, si1, sg, sw0, sw1):
        wid = lax.axis_index("s") * NC + lax.axis_index("c")
        w0 = pl.multiple_of(wid * _PER_W, 8)

        def base(t):
            return pl.multiple_of(w0 + t * _GC, 8)

        def idx_start(b, idv, isv, sem):
            pltpu.async_copy(dst_h.at[pl.ds(b, _GC)], idv, sem)
            pltpu.async_copy(src_h.at[pl.ds(b, _GC)], isv, sem)

        def idx_wait(b, idv, isv, sem):
            pltpu.make_async_copy(dst_h.at[pl.ds(b, _GC)], idv, sem).wait()
            pltpu.make_async_copy(src_h.at[pl.ds(b, _GC)], isv, sem).wait()

        def out_wait(b, bi, bj, sem):
            pltpu.make_async_copy(bi, oi_h.at[pl.ds(b, _GC)], sem).wait()
            pltpu.make_async_copy(bj, oj_h.at[pl.ds(b, _GC)], sem).wait()

        def process(b, idv, isv, bi, bj, wsem):
            d1 = pltpu.async_copy(yi_h.at[idv], bi, sg)
            d2 = pltpu.async_copy(yj_h.at[isv], bj, sg)
            d1.wait()
            d2.wait()
            pltpu.async_copy(bi, oi_h.at[pl.ds(b, _GC)], wsem)
            pltpu.async_copy(bj, oj_h.at[pl.ds(b, _GC)], wsem)

        idx_start(base(0), id0, is0, si0)

        @pl.loop(0, _GP)
        def _(p):
            b0 = base(2 * p)
            b1 = base(2 * p + 1)
            idx_start(b1, id1, is1, si1)
            idx_wait(b0, id0, is0, si0)

            @pl.when(p > 0)
            def _():
                out_wait(b0, bi0, bj0, sw0)

            process(b0, id0, is0, bi0, bj0, sw0)

            @pl.when(p < _GP - 1)
            def _():
                idx_start(base(2 * p + 2), id0, is0, si0)

            idx_wait(b1, id1, is1, si1)

            @pl.when(p > 0)
            def _():
                out_wait(b1, bi1, bj1, sw1)

            process(b1, id1, is1, bi1, bj1, sw1)

        tb = pl.multiple_of(w0 + _GN * _GC, 8)
        out_wait(base(0), bi0, bj0, sw0)
        pltpu.sync_copy(dst_h.at[pl.ds(tb, _GT)], id_t)
        pltpu.sync_copy(src_h.at[pl.ds(tb, _GT)], is_t)
        d1 = pltpu.async_copy(yi_h.at[id_t], bi0.at[pl.ds(0, _GT), :], sg)
        d2 = pltpu.async_copy(yj_h.at[is_t], bj0.at[pl.ds(0, _GT), :], sg)
        d1.wait()
        d2.wait()
        pltpu.sync_copy(bi0.at[pl.ds(0, _GT), :], oi_h.at[pl.ds(tb, _GT)])
        pltpu.sync_copy(bj0.at[pl.ds(0, _GT), :], oj_h.at[pl.ds(tb, _GT)])
        out_wait(base(0), bi1, bj1, sw1)

    return k(yi, yj, dst, src)


_SC_CH = 128
_ROWS_A = 80
_ROWS_B = 50
_NP = 10240
_RPT = _NP // NS


def _segment_sum(m2s, dst2, zeros_half):
    @functools.partial(
        pl.kernel,
        out_type=jax.ShapeDtypeStruct((NC, _NP, DH), jnp.float32),
        mesh=_mesh(),
        scratch_types=[
            pltpu.VMEM((_ROWS_A, _SC_CH), jnp.int32),
            pltpu.VMEM((_SC_CH, DH), jnp.float32),
            pltpu.VMEM((_SC_CH, DH), jnp.float32),
            pltpu.VMEM_SHARED((_NP, DH), jnp.float32),
            pltpu.SemaphoreType.DMA,
            pltpu.SemaphoreType.DMA,
            pltpu.SemaphoreType.DMA,
        ],
    )
    def k(m2s_h, dst2_h, z_h, agg_h, idx_a, b0, b1, acc_s, sz, s0, s1):
        c = lax.axis_index("c")
        tid = lax.axis_index("s")
        pltpu.sync_copy(z_h.at[pl.ds(tid * _RPT, _RPT)],
                        acc_s.at[pl.ds(tid * _RPT, _RPT)])

        @pl.when(tid < NS - 1)
        def _():
            pltpu.sync_copy(dst2_h.at[pl.ds(tid * _ROWS_A, _ROWS_A), :], idx_a)

        @pl.when(tid == NS - 1)
        def _():
            pltpu.sync_copy(dst2_h.at[pl.ds((NS - 1) * _ROWS_A, _ROWS_B), :],
                            idx_a.at[pl.ds(0, _ROWS_B), :])

        plsc.subcore_barrier()

        row0 = tid * _ROWS_A

        def src_at(r):
            base = pl.multiple_of(
                lax.min((row0 + r) * _SC_CH, E - _SC_CH), 8)
            return m2s_h.at[c, pl.ds(base, _SC_CH), :]

        def pipeline(npairs):
            pltpu.async_copy(src_at(0), b0, s0)

            @pl.loop(0, npairs)
            def _(p):
                r0 = 2 * p
                r1 = r0 + 1
                pltpu.async_copy(src_at(r1), b1, s1)
                pltpu.make_async_copy(src_at(r0), b0, s0).wait()
                pltpu.sync_copy(b0, acc_s.at[idx_a.at[r0]], add=True)
                pltpu.async_copy(src_at(r0 + 2), b0, s0)
                pltpu.make_async_copy(src_at(r1), b1, s1).wait()
                pltpu.sync_copy(b1, acc_s.at[idx_a.at[r1]], add=True)

            pltpu.make_async_copy(src_at(2 * npairs), b0, s0).wait()

        @pl.when(tid < NS - 1)
        def _():
            pipeline(_ROWS_A // 2)

        @pl.when(tid == NS - 1)
        def _():
            pipeline(_ROWS_B // 2)

        plsc.subcore_barrier()
        pltpu.sync_copy(acc_s.at[pl.ds(tid * _RPT, _RPT)],
                        agg_h.at[c, pl.ds(tid * _RPT, _RPT), :])

    return k(m2s, dst2, zeros_half)


_BN = 2000
_BE = 1280


def _dot(a, b):
    return jnp.dot(a, b, preferred_element_type=jnp.float32)


def _node_proj(x, wxi, wxj):
    def body(x_r, wi_r, wj_r, yi_r, yj_r):
        xb = x_r[...]
        yi_r[...] = _dot(xb, wi_r[...])
        yj_r[...] = _dot(xb, wj_r[...])

    full = lambda s: pl.BlockSpec(s, lambda i: (0, 0))
    return pl.pallas_call(
        body,
        grid=(N // _BN,),
        in_specs=[pl.BlockSpec((_BN, D), lambda i: (i, 0)), full((D, D)), full((D, D))],
        out_specs=[pl.BlockSpec((_BN, D), lambda i: (i, 0))] * 2,
        out_shape=[jax.ShapeDtypeStruct((N, D), jnp.float32)] * 2,
    )(x, wxi, wxj)


def _message(yi_g, yj_g, amf, edge, ea, w4, we, wa1, wm2, wa2):
    def body(yi_r, yj_r, amf_r, edge_r, ea_r, w4_r, we_r, wa1_r, wm2_r, wa2_r, out_r):
        eab = ea_r[...]
        t = (yi_r[...] + yj_r[...]
             + _dot(amf_r[...], w4_r[...]) + _dot(edge_r[...], we_r[...]))
        m1 = _swish(t * _dot(eab, wa1_r[...]))
        m2 = _swish(_dot(m1, wm2_r[...]) * _dot(eab, wa2_r[...]))
        out_r[0] = m2[:, :DH]
        out_r[1] = m2[:, DH:]

    full = lambda s: pl.BlockSpec(s, lambda i: tuple(0 for _ in s))
    return pl.pallas_call(
        body,
        grid=(E // _BE,),
        in_specs=[
            pl.BlockSpec((_BE, D), lambda i: (i, 0)),
            pl.BlockSpec((_BE, D), lambda i: (i, 0)),
            pl.BlockSpec((_BE, 4), lambda i: (i, 0)),
            pl.BlockSpec((_BE, D), lambda i: (i, 0)),
            pl.BlockSpec((_BE, 16), lambda i: (i, 0)),
            full((4, D)), full((D, D)), full((16, D)), full((D, D)), full((16, D)),
        ],
        out_specs=pl.BlockSpec((NC, _BE, DH), lambda i: (0, i, 0)),
        out_shape=jax.ShapeDtypeStruct((NC, E, DH), jnp.float32),
    )(yi_g, yj_g, amf, edge, ea, w4, we, wa1, wm2, wa2)


def _node_update(x, agg3, na, wu1a, wu1b, wau1, wu2, wau2, we1a, we1b):
    def body(x_r, ag_r, na_r, wu1a_r, wu1b_r, wau1_r, wu2_r, wau2_r,
             we1a_r, we1b_r, xn_r, ai_r, aj_r):
        xb = x_r[...]
        nab = na_r[...]
        agg = jnp.concatenate([ag_r[0], ag_r[1]], axis=-1)
        u = _swish((_dot(xb, wu1a_r[...]) + _dot(agg, wu1b_r[...]))
                   * _dot(nab, wau1_r[...]))
        u = _dot(u, wu2_r[...]) * _dot(nab, wau2_r[...])
        xn = xb + u
        xn_r[...] = xn
        ai_r[...] = _dot(xn, we1a_r[...])
        aj_r[...] = _dot(xn, we1b_r[...])

    full = lambda s: pl.BlockSpec(s, lambda i: tuple(0 for _ in s))
    return pl.pallas_call(
        body,
        grid=(N // _BN,),
        in_specs=[
            pl.BlockSpec((_BN, D), lambda i: (i, 0)),
            pl.BlockSpec((NC, _BN, DH), lambda i: (0, i, 0)),
            pl.BlockSpec((_BN, 16), lambda i: (i, 0)),
            full((D, D)), full((D, D)), full((16, D)),
            full((D, D)), full((16, D)), full((D, D)), full((D, D)),
        ],
        out_specs=[pl.BlockSpec((_BN, D), lambda i: (i, 0))] * 3,
        out_shape=[jax.ShapeDtypeStruct((N, D), jnp.float32)] * 3,
    )(x, agg3, na, wu1a, wu1b, wau1, wu2, wau2, we1a, we1b)


def _edge_update(ai_g, aj_g, edge, ea, g, wae1, wg1a, wg2a, we2, wae2, wg1b, wg2b):
    def body(ai_r, aj_r, edge_r, ea_r, g_r, wae1_r, wg1a_r, wg2a_r, we2_r, wae2_r,
             wg1b_r, wg2b_r, out_r):
        eab = ea_r[...]
        gb = g_r[...]
        wa = _dot(_swish(_dot(gb, wg1a_r[...])), wg2a_r[...])
        e1 = _swish((ai_r[...] + aj_r[...]) * _dot(eab, wae1_r[...]) * wa)
        wb = _dot(_swish(_dot(gb, wg1b_r[...])), wg2b_r[...])
        e2 = _swish(_dot(e1, we2_r[...]) * _dot(eab, wae2_r[...]) * wb)
        out_r[...] = edge_r[...] + e2

    full = lambda s: pl.BlockSpec(s, lambda i: tuple(0 for _ in s))
    return pl.pallas_call(
        body,
        grid=(E // _BE,),
        in_specs=[
            pl.BlockSpec((_BE, D), lambda i: (i, 0)),
            pl.BlockSpec((_BE, D), lambda i: (i, 0)),
            pl.BlockSpec((_BE, D), lambda i: (i, 0)),
            pl.BlockSpec((_BE, 16), lambda i: (i, 0)),
            pl.BlockSpec((_BE, 128), lambda i: (i, 0)),
            full((16, D)), full((128, 64)), full((64, D)), full((D, D)),
            full((16, D)), full((128, 64)), full((64, D)),
        ],
        out_specs=pl.BlockSpec((_BE, D), lambda i: (i, 0)),
        out_shape=jax.ShapeDtypeStruct((E, D), jnp.float32),
    )(ai_g, aj_g, edge, ea, g, wae1, wg1a, wg2a, we2, wae2, wg1b, wg2b)


def kernel(x, edge, edge_index, edge_attr, node_attr, additional_message_features,
           edge_dist_gauss, W_m1, Wa_m1, W_m2, Wa_m2, W_u1, Wa_u1, W_u2, Wa_u2,
           W_e1, Wa_e1, Wg1a, Wg2a, W_e2, Wa_e2, Wg1b, Wg2b):
    src = edge_index[0]
    dst = edge_index[1]
    dst2 = dst.reshape(E // _SC_CH, _SC_CH)
    zeros_half = jnp.zeros((_NP, DH), dtype=jnp.float32)

    w4 = W_m1[:4]
    yi, yj = _node_proj(x, W_m1[4:4 + D], W_m1[4 + D:4 + 2 * D])
    yig, yjg = _gather_pair(yi, yj, dst, src)
    m2s = _message(yig, yjg, additional_message_features, edge, edge_attr,
                   w4, W_m1[4 + 2 * D:], Wa_m1, W_m2, Wa_m2)
    agg3 = _segment_sum(m2s, dst2, zeros_half)[:, :N, :]

    x_new, ai, aj = _node_update(x, agg3, node_attr, W_u1[:D], W_u1[D:],
                                 Wa_u1, W_u2, Wa_u2, W_e1[:D], W_e1[D:])

    aig, ajg = _gather_pair(ai, aj, dst, src)
    edge_new = _edge_update(aig, ajg, edge, edge_attr, edge_dist_gauss,
                            Wa_e1, Wg1a, Wg2a, W_e2, Wa_e2, Wg1b, Wg2b)
    return (x_new, edge_new)


# speedup vs baseline: 1.3905x; 1.0235x over previous
import functools

import jax
import jax.numpy as jnp
from jax import lax
from jax.experimental import pallas as pl
from jax.experimental.pallas import tpu as pltpu
from jax.experimental.pallas import tpu_sc as plsc

N = 10000
E = 160000
D = 256
DH = 128

NC = 2
NS = 16
NW = NC * NS


@functools.cache
def _mesh():
    return plsc.VectorSubcoreMesh(
        core_axis_name="c", subcore_axis_name="s", num_cores=NC, num_subcores=NS)


def _swish(v):
    return v * jax.nn.sigmoid(v)


_GC = 112
_PER_W = E // NW
_GN = _PER_W // _GC
_GP = _GN // 2
_GT = _PER_W - _GN * _GC


def _gather_pair(yi, yj, dst, src):
    @functools.partial(
        pl.kernel,
        out_type=[jax.ShapeDtypeStruct((E, D), jnp.float32)] * 2,
        mesh=_mesh(),
        scratch_types=[
            pltpu.VMEM((_GC,), jnp.int32),
            pltpu.VMEM((_GC,), jnp.int32),
            pltpu.VMEM((_GC,), jnp.int32),
            pltpu.VMEM((_GC,), jnp.int32),
            pltpu.VMEM((_GC, D), jnp.float32),
            pltpu.VMEM((_GC, D), jnp.float32),
            pltpu.VMEM((_GC, D), jnp.float32),
            pltpu.VMEM((_GC, D), jnp.float32),
            pltpu.VMEM((_GT,), jnp.int32),
            pltpu.VMEM((_GT,), jnp.int32),
            pltpu.SemaphoreType.DMA,
            pltpu.SemaphoreType.DMA,
            pltpu.SemaphoreType.DMA,
            pltpu.SemaphoreType.DMA,
            pltpu.SemaphoreType.DMA,
        ],
    )
    def k(yi_h, yj_h, dst_h, src_h, oi_h, oj_h,
          id0, is0, id1, is1, bi0, bj0, bi1, bj1, id_t, is_t,
          si0, si1, sg, sw0, sw1):
        wid = lax.axis_index("s") * NC + lax.axis_index("c")
        w0 = pl.multiple_of(wid * _PER_W, 8)

        def base(t):
            return pl.multiple_of(w0 + t * _GC, 8)

        def idx_start(b, idv, isv, sem):
            pltpu.async_copy(dst_h.at[pl.ds(b, _GC)], idv, sem)
            pltpu.async_copy(src_h.at[pl.ds(b, _GC)], isv, sem)

        def idx_wait(b, idv, isv, sem):
            pltpu.make_async_copy(dst_h.at[pl.ds(b, _GC)], idv, sem).wait()
            pltpu.make_async_copy(src_h.at[pl.ds(b, _GC)], isv, sem).wait()

        def out_wait(b, bi, bj, sem):
            pltpu.make_async_copy(bi, oi_h.at[pl.ds(b, _GC)], sem).wait()
            pltpu.make_async_copy(bj, oj_h.at[pl.ds(b, _GC)], sem).wait()

        def process(b, idv, isv, bi, bj, wsem):
            d1 = pltpu.async_copy(yi_h.at[idv], bi, sg)
            d2 = pltpu.async_copy(yj_h.at[isv], bj, sg)
            d1.wait()
            d2.wait()
            pltpu.async_copy(bi, oi_h.at[pl.ds(b, _GC)], wsem)
            pltpu.async_copy(bj, oj_h.at[pl.ds(b, _GC)], wsem)

        idx_start(base(0), id0, is0, si0)

        @pl.loop(0, _GP)
        def _(p):
            b0 = base(2 * p)
            b1 = base(2 * p + 1)
            idx_start(b1, id1, is1, si1)
            idx_wait(b0, id0, is0, si0)

            @pl.when(p > 0)
            def _():
                out_wait(b0, bi0, bj0, sw0)

            process(b0, id0, is0, bi0, bj0, sw0)

            @pl.when(p < _GP - 1)
            def _():
                idx_start(base(2 * p + 2), id0, is0, si0)

            idx_wait(b1, id1, is1, si1)

            @pl.when(p > 0)
            def _():
                out_wait(b1, bi1, bj1, sw1)

            process(b1, id1, is1, bi1, bj1, sw1)

        tb = pl.multiple_of(w0 + _GN * _GC, 8)
        out_wait(base(0), bi0, bj0, sw0)
        pltpu.sync_copy(dst_h.at[pl.ds(tb, _GT)], id_t)
        pltpu.sync_copy(src_h.at[pl.ds(tb, _GT)], is_t)
        d1 = pltpu.async_copy(yi_h.at[id_t], bi0.at[pl.ds(0, _GT), :], sg)
        d2 = pltpu.async_copy(yj_h.at[is_t], bj0.at[pl.ds(0, _GT), :], sg)
        d1.wait()
        d2.wait()
        pltpu.sync_copy(bi0.at[pl.ds(0, _GT), :], oi_h.at[pl.ds(tb, _GT)])
        pltpu.sync_copy(bj0.at[pl.ds(0, _GT), :], oj_h.at[pl.ds(tb, _GT)])
        out_wait(base(0), bi1, bj1, sw1)

    return k(yi, yj, dst, src)


_SC_CH = 128
_ROWS_A = 80
_ROWS_B = 50
_NP = 10240
_RPT = _NP // NS


def _segment_sum(m2s, dst2, zeros_half):
    @functools.partial(
        pl.kernel,
        out_type=jax.ShapeDtypeStruct((NC, _NP, DH), jnp.float32),
        mesh=_mesh(),
        scratch_types=[
            pltpu.VMEM((_ROWS_A, _SC_CH), jnp.int32),
            pltpu.VMEM((_SC_CH, DH), jnp.float32),
            pltpu.VMEM((_SC_CH, DH), jnp.float32),
            pltpu.VMEM_SHARED((_NP, DH), jnp.float32),
            pltpu.SemaphoreType.DMA,
            pltpu.SemaphoreType.DMA,
            pltpu.SemaphoreType.DMA,
        ],
    )
    def k(m2s_h, dst2_h, z_h, agg_h, idx_a, b0, b1, acc_s, sz, s0, s1):
        c = lax.axis_index("c")
        tid = lax.axis_index("s")
        pltpu.sync_copy(z_h.at[pl.ds(tid * _RPT, _RPT)],
                        acc_s.at[pl.ds(tid * _RPT, _RPT)])

        @pl.when(tid < NS - 1)
        def _():
            pltpu.sync_copy(dst2_h.at[pl.ds(tid * _ROWS_A, _ROWS_A), :], idx_a)

        @pl.when(tid == NS - 1)
        def _():
            pltpu.sync_copy(dst2_h.at[pl.ds((NS - 1) * _ROWS_A, _ROWS_B), :],
                            idx_a.at[pl.ds(0, _ROWS_B), :])

        plsc.subcore_barrier()

        row0 = tid * _ROWS_A

        def src_at(r):
            base = pl.multiple_of(
                lax.min((row0 + r) * _SC_CH, E - _SC_CH), 8)
            return m2s_h.at[c, pl.ds(base, _SC_CH), :]

        def pipeline(npairs):
            pltpu.async_copy(src_at(0), b0, s0)

            @pl.loop(0, npairs)
            def _(p):
                r0 = 2 * p
                r1 = r0 + 1
                pltpu.async_copy(src_at(r1), b1, s1)
                pltpu.make_async_copy(src_at(r0), b0, s0).wait()
                pltpu.sync_copy(b0, acc_s.at[idx_a.at[r0]], add=True)
                pltpu.async_copy(src_at(r0 + 2), b0, s0)
                pltpu.make_async_copy(src_at(r1), b1, s1).wait()
                pltpu.sync_copy(b1, acc_s.at[idx_a.at[r1]], add=True)

            pltpu.make_async_copy(src_at(2 * npairs), b0, s0).wait()

        @pl.when(tid < NS - 1)
        def _():
            pipeline(_ROWS_A // 2)

        @pl.when(tid == NS - 1)
        def _():
            pipeline(_ROWS_B // 2)

        plsc.subcore_barrier()
        pltpu.sync_copy(acc_s.at[pl.ds(tid * _RPT, _RPT)],
                        agg_h.at[c, pl.ds(tid * _RPT, _RPT), :])

    return k(m2s, dst2, zeros_half)


_BN = 2000
_BE = 1600


def _dot(a, b):
    return jnp.dot(a, b, preferred_element_type=jnp.float32)


def _node_proj(x, wxi, wxj):
    def body(x_r, wi_r, wj_r, yi_r, yj_r):
        xb = x_r[...]
        yi_r[...] = _dot(xb, wi_r[...])
        yj_r[...] = _dot(xb, wj_r[...])

    full = lambda s: pl.BlockSpec(s, lambda i: (0, 0))
    return pl.pallas_call(
        body,
        grid=(N // _BN,),
        in_specs=[pl.BlockSpec((_BN, D), lambda i: (i, 0)), full((D, D)), full((D, D))],
        out_specs=[pl.BlockSpec((_BN, D), lambda i: (i, 0))] * 2,
        out_shape=[jax.ShapeDtypeStruct((N, D), jnp.float32)] * 2,
    )(x, wxi, wxj)


def _message(yi_g, yj_g, amf, edge, ea, w4, we, wa1, wm2, wa2):
    def body(yi_r, yj_r, amf_r, edge_r, ea_r, w4_r, we_r, wa1_r, wm2_r, wa2_r, out_r):
        eab = ea_r[...]
        t = (yi_r[...] + yj_r[...]
             + _dot(amf_r[...], w4_r[...]) + _dot(edge_r[...], we_r[...]))
        m1 = _swish(t * _dot(eab, wa1_r[...]))
        m2 = _swish(_dot(m1, wm2_r[...]) * _dot(eab, wa2_r[...]))
        out_r[0] = m2[:, :DH]
        out_r[1] = m2[:, DH:]

    full = lambda s: pl.BlockSpec(s, lambda i: tuple(0 for _ in s))
    return pl.pallas_call(
        body,
        grid=(E // _BE,),
        in_specs=[
            pl.BlockSpec((_BE, D), lambda i: (i, 0)),
            pl.BlockSpec((_BE, D), lambda i: (i, 0)),
            pl.BlockSpec((_BE, 4), lambda i: (i, 0)),
            pl.BlockSpec((_BE, D), lambda i: (i, 0)),
            pl.BlockSpec((_BE, 16), lambda i: (i, 0)),
            full((4, D)), full((D, D)), full((16, D)), full((D, D)), full((16, D)),
        ],
        out_specs=pl.BlockSpec((NC, _BE, DH), lambda i: (0, i, 0)),
        out_shape=jax.ShapeDtypeStruct((NC, E, DH), jnp.float32),
    )(yi_g, yj_g, amf, edge, ea, w4, we, wa1, wm2, wa2)


def _node_update(x, agg3, na, wu1a, wu1b, wau1, wu2, wau2, we1a, we1b):
    def body(x_r, ag_r, na_r, wu1a_r, wu1b_r, wau1_r, wu2_r, wau2_r,
             we1a_r, we1b_r, xn_r, ai_r, aj_r):
        xb = x_r[...]
        nab = na_r[...]
        agg = jnp.concatenate([ag_r[0], ag_r[1]], axis=-1)
        u = _swish((_dot(xb, wu1a_r[...]) + _dot(agg, wu1b_r[...]))
                   * _dot(nab, wau1_r[...]))
        u = _dot(u, wu2_r[...]) * _dot(nab, wau2_r[...])
        xn = xb + u
        xn_r[...] = xn
        ai_r[...] = _dot(xn, we1a_r[...])
        aj_r[...] = _dot(xn, we1b_r[...])

    full = lambda s: pl.BlockSpec(s, lambda i: tuple(0 for _ in s))
    return pl.pallas_call(
        body,
        grid=(N // _BN,),
        in_specs=[
            pl.BlockSpec((_BN, D), lambda i: (i, 0)),
            pl.BlockSpec((NC, _BN, DH), lambda i: (0, i, 0)),
            pl.BlockSpec((_BN, 16), lambda i: (i, 0)),
            full((D, D)), full((D, D)), full((16, D)),
            full((D, D)), full((16, D)), full((D, D)), full((D, D)),
        ],
        out_specs=[pl.BlockSpec((_BN, D), lambda i: (i, 0))] * 3,
        out_shape=[jax.ShapeDtypeStruct((N, D), jnp.float32)] * 3,
    )(x, agg3, na, wu1a, wu1b, wau1, wu2, wau2, we1a, we1b)


def _edge_update(ai_g, aj_g, edge, ea, g, wae1, wg1a, wg2a, we2, wae2, wg1b, wg2b):
    def body(ai_r, aj_r, edge_r, ea_r, g_r, wae1_r, wg1a_r, wg2a_r, we2_r, wae2_r,
             wg1b_r, wg2b_r, out_r):
        eab = ea_r[...]
        gb = g_r[...]
        wa = _dot(_swish(_dot(gb, wg1a_r[...])), wg2a_r[...])
        e1 = _swish((ai_r[...] + aj_r[...]) * _dot(eab, wae1_r[...]) * wa)
        wb = _dot(_swish(_dot(gb, wg1b_r[...])), wg2b_r[...])
        e2 = _swish(_dot(e1, we2_r[...]) * _dot(eab, wae2_r[...]) * wb)
        out_r[...] = edge_r[...] + e2

    full = lambda s: pl.BlockSpec(s, lambda i: tuple(0 for _ in s))
    return pl.pallas_call(
        body,
        grid=(E // _BE,),
        in_specs=[
            pl.BlockSpec((_BE, D), lambda i: (i, 0)),
            pl.BlockSpec((_BE, D), lambda i: (i, 0)),
            pl.BlockSpec((_BE, D), lambda i: (i, 0)),
            pl.BlockSpec((_BE, 16), lambda i: (i, 0)),
            pl.BlockSpec((_BE, 128), lambda i: (i, 0)),
            full((16, D)), full((128, 64)), full((64, D)), full((D, D)),
            full((16, D)), full((128, 64)), full((64, D)),
        ],
        out_specs=pl.BlockSpec((_BE, D), lambda i: (i, 0)),
        out_shape=jax.ShapeDtypeStruct((E, D), jnp.float32),
    )(ai_g, aj_g, edge, ea, g, wae1, wg1a, wg2a, we2, wae2, wg1b, wg2b)


def kernel(x, edge, edge_index, edge_attr, node_attr, additional_message_features,
           edge_dist_gauss, W_m1, Wa_m1, W_m2, Wa_m2, W_u1, Wa_u1, W_u2, Wa_u2,
           W_e1, Wa_e1, Wg1a, Wg2a, W_e2, Wa_e2, Wg1b, Wg2b):
    src = edge_index[0]
    dst = edge_index[1]
    dst2 = dst.reshape(E // _SC_CH, _SC_CH)
    zeros_half = jnp.zeros((_NP, DH), dtype=jnp.float32)

    w4 = W_m1[:4]
    yi, yj = _node_proj(x, W_m1[4:4 + D], W_m1[4 + D:4 + 2 * D])
    yig, yjg = _gather_pair(yi, yj, dst, src)
    m2s = _message(yig, yjg, additional_message_features, edge, edge_attr,
                   w4, W_m1[4 + 2 * D:], Wa_m1, W_m2, Wa_m2)
    agg3 = _segment_sum(m2s, dst2, zeros_half)[:, :N, :]

    x_new, ai, aj = _node_update(x, agg3, node_attr, W_u1[:D], W_u1[D:],
                                 Wa_u1, W_u2, Wa_u2, W_e1[:D], W_e1[D:])

    aig, ajg = _gather_pair(ai, aj, dst, src)
    edge_new = _edge_update(aig, ajg, edge, edge_attr, edge_dist_gauss,
                            Wa_e1, Wg1a, Wg2a, W_e2, Wa_e2, Wg1b, Wg2b)
    return (x_new, edge_new)


# speedup vs baseline: 1.4037x; 1.0095x over previous
import functools

import jax
import jax.numpy as jnp
from jax import lax
from jax.experimental import pallas as pl
from jax.experimental.pallas import tpu as pltpu
from jax.experimental.pallas import tpu_sc as plsc

N = 10000
E = 160000
D = 256
DH = 128

NC = 2
NS = 16
NW = NC * NS


@functools.cache
def _mesh():
    return plsc.VectorSubcoreMesh(
        core_axis_name="c", subcore_axis_name="s", num_cores=NC, num_subcores=NS)


def _swish(v):
    return v * jax.nn.sigmoid(v)


_GC = 112
_PER_W = E // NW
_GN = _PER_W // _GC
_GP = _GN // 2
_GT = _PER_W - _GN * _GC


def _gather_pair(yi, yj, dst, src):
    @functools.partial(
        pl.kernel,
        out_type=[jax.ShapeDtypeStruct((E, D), jnp.float32)] * 2,
        mesh=_mesh(),
        scratch_types=[
            pltpu.VMEM((_GC,), jnp.int32),
            pltpu.VMEM((_GC,), jnp.int32),
            pltpu.VMEM((_GC,), jnp.int32),
            pltpu.VMEM((_GC,), jnp.int32),
            pltpu.VMEM((_GC, D), jnp.float32),
            pltpu.VMEM((_GC, D), jnp.float32),
            pltpu.VMEM((_GC, D), jnp.float32),
            pltpu.VMEM((_GC, D), jnp.float32),
            pltpu.VMEM((_GT,), jnp.int32),
            pltpu.VMEM((_GT,), jnp.int32),
            pltpu.SemaphoreType.DMA,
            pltpu.SemaphoreType.DMA,
            pltpu.SemaphoreType.DMA,
            pltpu.SemaphoreType.DMA,
            pltpu.SemaphoreType.DMA,
        ],
    )
    def k(yi_h, yj_h, dst_h, src_h, oi_h, oj_h,
          id0, is0, id1, is1, bi0, bj0, bi1, bj1, id_t, is_t,
          si0, si1, sg, sw0, sw1):
        wid = lax.axis_index("s") * NC + lax.axis_index("c")
        w0 = pl.multiple_of(wid * _PER_W, 8)

        def base(t):
            return pl.multiple_of(w0 + t * _GC, 8)

        def idx_start(b, idv, isv, sem):
            pltpu.async_copy(dst_h.at[pl.ds(b, _GC)], idv, sem)
            pltpu.async_copy(src_h.at[pl.ds(b, _GC)], isv, sem)

        def idx_wait(b, idv, isv, sem):
            pltpu.make_async_copy(dst_h.at[pl.ds(b, _GC)], idv, sem).wait()
            pltpu.make_async_copy(src_h.at[pl.ds(b, _GC)], isv, sem).wait()

        def out_wait(b, bi, bj, sem):
            pltpu.make_async_copy(bi, oi_h.at[pl.ds(b, _GC)], sem).wait()
            pltpu.make_async_copy(bj, oj_h.at[pl.ds(b, _GC)], sem).wait()

        def process(b, idv, isv, bi, bj, wsem):
            d1 = pltpu.async_copy(yi_h.at[idv], bi, sg)
            d2 = pltpu.async_copy(yj_h.at[isv], bj, sg)
            d1.wait()
            d2.wait()
            pltpu.async_copy(bi, oi_h.at[pl.ds(b, _GC)], wsem)
            pltpu.async_copy(bj, oj_h.at[pl.ds(b, _GC)], wsem)

        idx_start(base(0), id0, is0, si0)

        @pl.loop(0, _GP)
        def _(p):
            b0 = base(2 * p)
            b1 = base(2 * p + 1)
            idx_start(b1, id1, is1, si1)
            idx_wait(b0, id0, is0, si0)

            @pl.when(p > 0)
            def _():
                out_wait(b0, bi0, bj0, sw0)

            process(b0, id0, is0, bi0, bj0, sw0)

            @pl.when(p < _GP - 1)
            def _():
                idx_start(base(2 * p + 2), id0, is0, si0)

            idx_wait(b1, id1, is1, si1)

            @pl.when(p > 0)
            def _():
                out_wait(b1, bi1, bj1, sw1)

            process(b1, id1, is1, bi1, bj1, sw1)

        tb = pl.multiple_of(w0 + _GN * _GC, 8)
        out_wait(base(0), bi0, bj0, sw0)
        pltpu.sync_copy(dst_h.at[pl.ds(tb, _GT)], id_t)
        pltpu.sync_copy(src_h.at[pl.ds(tb, _GT)], is_t)
        d1 = pltpu.async_copy(yi_h.at[id_t], bi0.at[pl.ds(0, _GT), :], sg)
        d2 = pltpu.async_copy(yj_h.at[is_t], bj0.at[pl.ds(0, _GT), :], sg)
        d1.wait()
        d2.wait()
        pltpu.sync_copy(bi0.at[pl.ds(0, _GT), :], oi_h.at[pl.ds(tb, _GT)])
        pltpu.sync_copy(bj0.at[pl.ds(0, _GT), :], oj_h.at[pl.ds(tb, _GT)])
        out_wait(base(0), bi1, bj1, sw1)

    return k(yi, yj, dst, src)


_SC_CH = 128
_ROWS_A = 80
_ROWS_B = 50
_NP = 10240
_RPT = _NP // NS


def _segment_sum(m2s, dst2, zeros_half):
    @functools.partial(
        pl.kernel,
        out_type=jax.ShapeDtypeStruct((NC, _NP, DH), jnp.float32),
        mesh=_mesh(),
        scratch_types=[
            pltpu.VMEM((_ROWS_A, _SC_CH), jnp.int32),
            pltpu.VMEM((_SC_CH, DH), jnp.float32),
            pltpu.VMEM((_SC_CH, DH), jnp.float32),
            pltpu.VMEM_SHARED((_NP, DH), jnp.float32),
            pltpu.SemaphoreType.DMA,
            pltpu.SemaphoreType.DMA,
            pltpu.SemaphoreType.DMA,
        ],
    )
    def k(m2s_h, dst2_h, z_h, agg_h, idx_a, b0, b1, acc_s, sz, s0, s1):
        c = lax.axis_index("c")
        tid = lax.axis_index("s")
        pltpu.sync_copy(z_h.at[pl.ds(tid * _RPT, _RPT)],
                        acc_s.at[pl.ds(tid * _RPT, _RPT)])

        @pl.when(tid < NS - 1)
        def _():
            pltpu.sync_copy(dst2_h.at[pl.ds(tid * _ROWS_A, _ROWS_A), :], idx_a)

        @pl.when(tid == NS - 1)
        def _():
            pltpu.sync_copy(dst2_h.at[pl.ds((NS - 1) * _ROWS_A, _ROWS_B), :],
                            idx_a.at[pl.ds(0, _ROWS_B), :])

        plsc.subcore_barrier()

        row0 = tid * _ROWS_A

        def src_at(r):
            base = pl.multiple_of(
                lax.min((row0 + r) * _SC_CH, E - _SC_CH), 8)
            return m2s_h.at[c, pl.ds(base, _SC_CH), :]

        def pipeline(npairs):
            pltpu.async_copy(src_at(0), b0, s0)

            @pl.loop(0, npairs)
            def _(p):
                r0 = 2 * p
                r1 = r0 + 1
                pltpu.async_copy(src_at(r1), b1, s1)
                pltpu.make_async_copy(src_at(r0), b0, s0).wait()
                pltpu.sync_copy(b0, acc_s.at[idx_a.at[r0]], add=True)
                pltpu.async_copy(src_at(r0 + 2), b0, s0)
                pltpu.make_async_copy(src_at(r1), b1, s1).wait()
                pltpu.sync_copy(b1, acc_s.at[idx_a.at[r1]], add=True)

            pltpu.make_async_copy(src_at(2 * npairs), b0, s0).wait()

        @pl.when(tid < NS - 1)
        def _():
            pipeline(_ROWS_A // 2)

        @pl.when(tid == NS - 1)
        def _():
            pipeline(_ROWS_B // 2)

        plsc.subcore_barrier()
        pltpu.sync_copy(acc_s.at[pl.ds(tid * _RPT, _RPT)],
                        agg_h.at[c, pl.ds(tid * _RPT, _RPT), :])

    return k(m2s, dst2, zeros_half)


_BN = 2000
_BE = 2000


def _dot(a, b):
    return jnp.dot(a, b, preferred_element_type=jnp.float32)


def _node_proj(x, wxi, wxj):
    def body(x_r, wi_r, wj_r, yi_r, yj_r):
        xb = x_r[...]
        yi_r[...] = _dot(xb, wi_r[...])
        yj_r[...] = _dot(xb, wj_r[...])

    full = lambda s: pl.BlockSpec(s, lambda i: (0, 0))
    return pl.pallas_call(
        body,
        grid=(N // _BN,),
        in_specs=[pl.BlockSpec((_BN, D), lambda i: (i, 0)), full((D, D)), full((D, D))],
        out_specs=[pl.BlockSpec((_BN, D), lambda i: (i, 0))] * 2,
        out_shape=[jax.ShapeDtypeStruct((N, D), jnp.float32)] * 2,
    )(x, wxi, wxj)


def _message(yi_g, yj_g, amf, edge, ea, w4, we, wa1, wm2, wa2):
    def body(yi_r, yj_r, amf_r, edge_r, ea_r, w4_r, we_r, wa1_r, wm2_r, wa2_r, out_r):
        eab = ea_r[...]
        t = (yi_r[...] + yj_r[...]
             + _dot(amf_r[...], w4_r[...]) + _dot(edge_r[...], we_r[...]))
        m1 = _swish(t * _dot(eab, wa1_r[...]))
        m2 = _swish(_dot(m1, wm2_r[...]) * _dot(eab, wa2_r[...]))
        out_r[0] = m2[:, :DH]
        out_r[1] = m2[:, DH:]

    full = lambda s: pl.BlockSpec(s, lambda i: tuple(0 for _ in s))
    return pl.pallas_call(
        body,
        grid=(E // _BE,),
        in_specs=[
            pl.BlockSpec((_BE, D), lambda i: (i, 0)),
            pl.BlockSpec((_BE, D), lambda i: (i, 0)),
            pl.BlockSpec((_BE, 4), lambda i: (i, 0)),
            pl.BlockSpec((_BE, D), lambda i: (i, 0)),
            pl.BlockSpec((_BE, 16), lambda i: (i, 0)),
            full((4, D)), full((D, D)), full((16, D)), full((D, D)), full((16, D)),
        ],
        out_specs=pl.BlockSpec((NC, _BE, DH), lambda i: (0, i, 0)),
        out_shape=jax.ShapeDtypeStruct((NC, E, DH), jnp.float32),
    )(yi_g, yj_g, amf, edge, ea, w4, we, wa1, wm2, wa2)


def _node_update(x, agg3, na, wu1a, wu1b, wau1, wu2, wau2, we1a, we1b):
    def body(x_r, ag_r, na_r, wu1a_r, wu1b_r, wau1_r, wu2_r, wau2_r,
             we1a_r, we1b_r, xn_r, ai_r, aj_r):
        xb = x_r[...]
        nab = na_r[...]
        agg = jnp.concatenate([ag_r[0], ag_r[1]], axis=-1)
        u = _swish((_dot(xb, wu1a_r[...]) + _dot(agg, wu1b_r[...]))
                   * _dot(nab, wau1_r[...]))
        u = _dot(u, wu2_r[...]) * _dot(nab, wau2_r[...])
        xn = xb + u
        xn_r[...] = xn
        ai_r[...] = _dot(xn, we1a_r[...])
        aj_r[...] = _dot(xn, we1b_r[...])

    full = lambda s: pl.BlockSpec(s, lambda i: tuple(0 for _ in s))
    return pl.pallas_call(
        body,
        grid=(N // _BN,),
        in_specs=[
            pl.BlockSpec((_BN, D), lambda i: (i, 0)),
            pl.BlockSpec((NC, _BN, DH), lambda i: (0, i, 0)),
            pl.BlockSpec((_BN, 16), lambda i: (i, 0)),
            full((D, D)), full((D, D)), full((16, D)),
            full((D, D)), full((16, D)), full((D, D)), full((D, D)),
        ],
        out_specs=[pl.BlockSpec((_BN, D), lambda i: (i, 0))] * 3,
        out_shape=[jax.ShapeDtypeStruct((N, D), jnp.float32)] * 3,
    )(x, agg3, na, wu1a, wu1b, wau1, wu2, wau2, we1a, we1b)


def _edge_update(ai_g, aj_g, edge, ea, g, wae1, wg1a, wg2a, we2, wae2, wg1b, wg2b):
    def body(ai_r, aj_r, edge_r, ea_r, g_r, wae1_r, wg1a_r, wg2a_r, we2_r, wae2_r,
             wg1b_r, wg2b_r, out_r):
        eab = ea_r[...]
        gb = g_r[...]
        wa = _dot(_swish(_dot(gb, wg1a_r[...])), wg2a_r[...])
        e1 = _swish((ai_r[...] + aj_r[...]) * _dot(eab, wae1_r[...]) * wa)
        wb = _dot(_swish(_dot(gb, wg1b_r[...])), wg2b_r[...])
        e2 = _swish(_dot(e1, we2_r[...]) * _dot(eab, wae2_r[...]) * wb)
        out_r[...] = edge_r[...] + e2

    full = lambda s: pl.BlockSpec(s, lambda i: tuple(0 for _ in s))
    return pl.pallas_call(
        body,
        grid=(E // _BE,),
        in_specs=[
            pl.BlockSpec((_BE, D), lambda i: (i, 0)),
            pl.BlockSpec((_BE, D), lambda i: (i, 0)),
            pl.BlockSpec((_BE, D), lambda i: (i, 0)),
            pl.BlockSpec((_BE, 16), lambda i: (i, 0)),
            pl.BlockSpec((_BE, 128), lambda i: (i, 0)),
            full((16, D)), full((128, 64)), full((64, D)), full((D, D)),
            full((16, D)), full((128, 64)), full((64, D)),
        ],
        out_specs=pl.BlockSpec((_BE, D), lambda i: (i, 0)),
        out_shape=jax.ShapeDtypeStruct((E, D), jnp.float32),
    )(ai_g, aj_g, edge, ea, g, wae1, wg1a, wg2a, we2, wae2, wg1b, wg2b)


def kernel(x, edge, edge_index, edge_attr, node_attr, additional_message_features,
           edge_dist_gauss, W_m1, Wa_m1, W_m2, Wa_m2, W_u1, Wa_u1, W_u2, Wa_u2,
           W_e1, Wa_e1, Wg1a, Wg2a, W_e2, Wa_e2, Wg1b, Wg2b):
    src = edge_index[0]
    dst = edge_index[1]
    dst2 = dst.reshape(E // _SC_CH, _SC_CH)
    zeros_half = jnp.zeros((_NP, DH), dtype=jnp.float32)

    w4 = W_m1[:4]
    yi, yj = _node_proj(x, W_m1[4:4 + D], W_m1[4 + D:4 + 2 * D])
    yig, yjg = _gather_pair(yi, yj, dst, src)
    m2s = _message(yig, yjg, additional_message_features, edge, edge_attr,
                   w4, W_m1[4 + 2 * D:], Wa_m1, W_m2, Wa_m2)
    agg3 = _segment_sum(m2s, dst2, zeros_half)[:, :N, :]

    x_new, ai, aj = _node_update(x, agg3, node_attr, W_u1[:D], W_u1[D:],
                                 Wa_u1, W_u2, Wa_u2, W_e1[:D], W_e1[D:])

    aig, ajg = _gather_pair(ai, aj, dst, src)
    edge_new = _edge_update(aig, ajg, edge, edge_attr, edge_dist_gauss,
                            Wa_e1, Wg1a, Wg2a, W_e2, Wa_e2, Wg1b, Wg2b)
    return (x_new, edge_new)


# speedup vs baseline: 1.4260x; 1.0159x over previous
import functools

import jax
import jax.numpy as jnp
from jax import lax
from jax.experimental import pallas as pl
from jax.experimental.pallas import tpu as pltpu
from jax.experimental.pallas import tpu_sc as plsc

N = 10000
E = 160000
D = 256
DH = 128

NC = 2
NS = 16
NW = NC * NS


@functools.cache
def _mesh():
    return plsc.VectorSubcoreMesh(
        core_axis_name="c", subcore_axis_name="s", num_cores=NC, num_subcores=NS)


def _swish(v):
    return v * jax.nn.sigmoid(v)


_GC = 112
_PER_W = E // NW
_GN = _PER_W // _GC
_GP = _GN // 2
_GT = _PER_W - _GN * _GC


def _gather_pair(yi, yj, dst, src):
    @functools.partial(
        pl.kernel,
        out_type=[jax.ShapeDtypeStruct((E, D), jnp.float32)] * 2,
        mesh=_mesh(),
        scratch_types=[
            pltpu.VMEM((_GC,), jnp.int32),
            pltpu.VMEM((_GC,), jnp.int32),
            pltpu.VMEM((_GC,), jnp.int32),
            pltpu.VMEM((_GC,), jnp.int32),
            pltpu.VMEM((_GC, D), jnp.float32),
            pltpu.VMEM((_GC, D), jnp.float32),
            pltpu.VMEM((_GC, D), jnp.float32),
            pltpu.VMEM((_GC, D), jnp.float32),
            pltpu.VMEM((_GT,), jnp.int32),
            pltpu.VMEM((_GT,), jnp.int32),
            pltpu.SemaphoreType.DMA,
            pltpu.SemaphoreType.DMA,
            pltpu.SemaphoreType.DMA,
            pltpu.SemaphoreType.DMA,
            pltpu.SemaphoreType.DMA,
        ],
    )
    def k(yi_h, yj_h, dst_h, src_h, oi_h, oj_h,
          id0, is0, id1, is1, bi0, bj0, bi1, bj1, id_t, is_t,
          si0, si1, sg, sw0, sw1):
        wid = lax.axis_index("s") * NC + lax.axis_index("c")
        w0 = pl.multiple_of(wid * _PER_W, 8)

        def base(t):
            return pl.multiple_of(w0 + t * _GC, 8)

        def idx_start(b, idv, isv, sem):
            pltpu.async_copy(dst_h.at[pl.ds(b, _GC)], idv, sem)
            pltpu.async_copy(src_h.at[pl.ds(b, _GC)], isv, sem)

        def idx_wait(b, idv, isv, sem):
            pltpu.make_async_copy(dst_h.at[pl.ds(b, _GC)], idv, sem).wait()
            pltpu.make_async_copy(src_h.at[pl.ds(b, _GC)], isv, sem).wait()

        def out_wait(b, bi, bj, sem):
            pltpu.make_async_copy(bi, oi_h.at[pl.ds(b, _GC)], sem).wait()
            pltpu.make_async_copy(bj, oj_h.at[pl.ds(b, _GC)], sem).wait()

        def process(b, idv, isv, bi, bj, wsem):
            d1 = pltpu.async_copy(yi_h.at[idv], bi, sg)
            d2 = pltpu.async_copy(yj_h.at[isv], bj, sg)
            d1.wait()
            d2.wait()
            pltpu.async_copy(bi, oi_h.at[pl.ds(b, _GC)], wsem)
            pltpu.async_copy(bj, oj_h.at[pl.ds(b, _GC)], wsem)

        idx_start(base(0), id0, is0, si0)

        @pl.loop(0, _GP)
        def _(p):
            b0 = base(2 * p)
            b1 = base(2 * p + 1)
            idx_start(b1, id1, is1, si1)
            idx_wait(b0, id0, is0, si0)

            @pl.when(p > 0)
            def _():
                out_wait(b0, bi0, bj0, sw0)

            process(b0, id0, is0, bi0, bj0, sw0)

            @pl.when(p < _GP - 1)
            def _():
                idx_start(base(2 * p + 2), id0, is0, si0)

            idx_wait(b1, id1, is1, si1)

            @pl.when(p > 0)
            def _():
                out_wait(b1, bi1, bj1, sw1)

            process(b1, id1, is1, bi1, bj1, sw1)

        tb = pl.multiple_of(w0 + _GN * _GC, 8)
        out_wait(base(0), bi0, bj0, sw0)
        pltpu.sync_copy(dst_h.at[pl.ds(tb, _GT)], id_t)
        pltpu.sync_copy(src_h.at[pl.ds(tb, _GT)], is_t)
        d1 = pltpu.async_copy(yi_h.at[id_t], bi0.at[pl.ds(0, _GT), :], sg)
        d2 = pltpu.async_copy(yj_h.at[is_t], bj0.at[pl.ds(0, _GT), :], sg)
        d1.wait()
        d2.wait()
        pltpu.sync_copy(bi0.at[pl.ds(0, _GT), :], oi_h.at[pl.ds(tb, _GT)])
        pltpu.sync_copy(bj0.at[pl.ds(0, _GT), :], oj_h.at[pl.ds(tb, _GT)])
        out_wait(base(0), bi1, bj1, sw1)

    return k(yi, yj, dst, src)


_SC_CH = 128
_ROWS_A = 80
_ROWS_B = 50
_NP = 10240
_RPT = _NP // NS


def _segment_sum(m2s, dst2, zeros_half):
    @functools.partial(
        pl.kernel,
        out_type=jax.ShapeDtypeStruct((NC, _NP, DH), jnp.float32),
        mesh=_mesh(),
        scratch_types=[
            pltpu.VMEM((_ROWS_A, _SC_CH), jnp.int32),
            pltpu.VMEM((_SC_CH, DH), jnp.float32),
            pltpu.VMEM((_SC_CH, DH), jnp.float32),
            pltpu.VMEM_SHARED((_NP, DH), jnp.float32),
            pltpu.SemaphoreType.DMA,
            pltpu.SemaphoreType.DMA,
            pltpu.SemaphoreType.DMA,
        ],
    )
    def k(m2s_h, dst2_h, z_h, agg_h, idx_a, b0, b1, acc_s, sz, s0, s1):
        c = lax.axis_index("c")
        tid = lax.axis_index("s")
        pltpu.sync_copy(z_h.at[pl.ds(tid * _RPT, _RPT)],
                        acc_s.at[pl.ds(tid * _RPT, _RPT)])

        @pl.when(tid < NS - 1)
        def _():
            pltpu.sync_copy(dst2_h.at[pl.ds(tid * _ROWS_A, _ROWS_A), :], idx_a)

        @pl.when(tid == NS - 1)
        def _():
            pltpu.sync_copy(dst2_h.at[pl.ds((NS - 1) * _ROWS_A, _ROWS_B), :],
                            idx_a.at[pl.ds(0, _ROWS_B), :])

        plsc.subcore_barrier()

        row0 = tid * _ROWS_A

        def src_at(r):
            base = pl.multiple_of(
                lax.min((row0 + r) * _SC_CH, E - _SC_CH), 8)
            return m2s_h.at[c, pl.ds(base, _SC_CH), :]

        def pipeline(npairs):
            pltpu.async_copy(src_at(0), b0, s0)

            @pl.loop(0, npairs)
            def _(p):
                r0 = 2 * p
                r1 = r0 + 1
                pltpu.async_copy(src_at(r1), b1, s1)
                pltpu.make_async_copy(src_at(r0), b0, s0).wait()
                pltpu.sync_copy(b0, acc_s.at[idx_a.at[r0]], add=True)
                pltpu.async_copy(src_at(r0 + 2), b0, s0)
                pltpu.make_async_copy(src_at(r1), b1, s1).wait()
                pltpu.sync_copy(b1, acc_s.at[idx_a.at[r1]], add=True)

            pltpu.make_async_copy(src_at(2 * npairs), b0, s0).wait()

        @pl.when(tid < NS - 1)
        def _():
            pipeline(_ROWS_A // 2)

        @pl.when(tid == NS - 1)
        def _():
            pipeline(_ROWS_B // 2)

        plsc.subcore_barrier()
        pltpu.sync_copy(acc_s.at[pl.ds(tid * _RPT, _RPT)],
                        agg_h.at[c, pl.ds(tid * _RPT, _RPT), :])

    return k(m2s, dst2, zeros_half)


_BN = 2000
_BE = 3200


def _dot(a, b):
    return jnp.dot(a, b, preferred_element_type=jnp.float32)


def _node_proj(x, wxi, wxj):
    def body(x_r, wi_r, wj_r, yi_r, yj_r):
        xb = x_r[...]
        yi_r[...] = _dot(xb, wi_r[...])
        yj_r[...] = _dot(xb, wj_r[...])

    full = lambda s: pl.BlockSpec(s, lambda i: (0, 0))
    return pl.pallas_call(
        body,
        grid=(N // _BN,),
        in_specs=[pl.BlockSpec((_BN, D), lambda i: (i, 0)), full((D, D)), full((D, D))],
        out_specs=[pl.BlockSpec((_BN, D), lambda i: (i, 0))] * 2,
        out_shape=[jax.ShapeDtypeStruct((N, D), jnp.float32)] * 2,
    )(x, wxi, wxj)


def _message(yi_g, yj_g, amf, edge, ea, w4, we, wa1, wm2, wa2):
    def body(yi_r, yj_r, amf_r, edge_r, ea_r, w4_r, we_r, wa1_r, wm2_r, wa2_r, out_r):
        eab = ea_r[...]
        t = (yi_r[...] + yj_r[...]
             + _dot(amf_r[...], w4_r[...]) + _dot(edge_r[...], we_r[...]))
        m1 = _swish(t * _dot(eab, wa1_r[...]))
        m2 = _swish(_dot(m1, wm2_r[...]) * _dot(eab, wa2_r[...]))
        out_r[0] = m2[:, :DH]
        out_r[1] = m2[:, DH:]

    full = lambda s: pl.BlockSpec(s, lambda i: tuple(0 for _ in s))
    return pl.pallas_call(
        body,
        grid=(E // _BE,),
        in_specs=[
            pl.BlockSpec((_BE, D), lambda i: (i, 0)),
            pl.BlockSpec((_BE, D), lambda i: (i, 0)),
            pl.BlockSpec((_BE, 4), lambda i: (i, 0)),
            pl.BlockSpec((_BE, D), lambda i: (i, 0)),
            pl.BlockSpec((_BE, 16), lambda i: (i, 0)),
            full((4, D)), full((D, D)), full((16, D)), full((D, D)), full((16, D)),
        ],
        out_specs=pl.BlockSpec((NC, _BE, DH), lambda i: (0, i, 0)),
        out_shape=jax.ShapeDtypeStruct((NC, E, DH), jnp.float32),
    )(yi_g, yj_g, amf, edge, ea, w4, we, wa1, wm2, wa2)


def _node_update(x, agg3, na, wu1a, wu1b, wau1, wu2, wau2, we1a, we1b):
    def body(x_r, ag_r, na_r, wu1a_r, wu1b_r, wau1_r, wu2_r, wau2_r,
             we1a_r, we1b_r, xn_r, ai_r, aj_r):
        xb = x_r[...]
        nab = na_r[...]
        agg = jnp.concatenate([ag_r[0], ag_r[1]], axis=-1)
        u = _swish((_dot(xb, wu1a_r[...]) + _dot(agg, wu1b_r[...]))
                   * _dot(nab, wau1_r[...]))
        u = _dot(u, wu2_r[...]) * _dot(nab, wau2_r[...])
        xn = xb + u
        xn_r[...] = xn
        ai_r[...] = _dot(xn, we1a_r[...])
        aj_r[...] = _dot(xn, we1b_r[...])

    full = lambda s: pl.BlockSpec(s, lambda i: tuple(0 for _ in s))
    return pl.pallas_call(
        body,
        grid=(N // _BN,),
        in_specs=[
            pl.BlockSpec((_BN, D), lambda i: (i, 0)),
            pl.BlockSpec((NC, _BN, DH), lambda i: (0, i, 0)),
            pl.BlockSpec((_BN, 16), lambda i: (i, 0)),
            full((D, D)), full((D, D)), full((16, D)),
            full((D, D)), full((16, D)), full((D, D)), full((D, D)),
        ],
        out_specs=[pl.BlockSpec((_BN, D), lambda i: (i, 0))] * 3,
        out_shape=[jax.ShapeDtypeStruct((N, D), jnp.float32)] * 3,
    )(x, agg3, na, wu1a, wu1b, wau1, wu2, wau2, we1a, we1b)


def _edge_update(ai_g, aj_g, edge, ea, g, wae1, wg1a, wg2a, we2, wae2, wg1b, wg2b):
    def body(ai_r, aj_r, edge_r, ea_r, g_r, wae1_r, wg1a_r, wg2a_r, we2_r, wae2_r,
             wg1b_r, wg2b_r, out_r):
        eab = ea_r[...]
        gb = g_r[...]
        wa = _dot(_swish(_dot(gb, wg1a_r[...])), wg2a_r[...])
        e1 = _swish((ai_r[...] + aj_r[...]) * _dot(eab, wae1_r[...]) * wa)
        wb = _dot(_swish(_dot(gb, wg1b_r[...])), wg2b_r[...])
        e2 = _swish(_dot(e1, we2_r[...]) * _dot(eab, wae2_r[...]) * wb)
        out_r[...] = edge_r[...] + e2

    full = lambda s: pl.BlockSpec(s, lambda i: tuple(0 for _ in s))
    return pl.pallas_call(
        body,
        grid=(E // _BE,),
        in_specs=[
            pl.BlockSpec((_BE, D), lambda i: (i, 0)),
            pl.BlockSpec((_BE, D), lambda i: (i, 0)),
            pl.BlockSpec((_BE, D), lambda i: (i, 0)),
            pl.BlockSpec((_BE, 16), lambda i: (i, 0)),
            pl.BlockSpec((_BE, 128), lambda i: (i, 0)),
            full((16, D)), full((128, 64)), full((64, D)), full((D, D)),
            full((16, D)), full((128, 64)), full((64, D)),
        ],
        out_specs=pl.BlockSpec((_BE, D), lambda i: (i, 0)),
        out_shape=jax.ShapeDtypeStruct((E, D), jnp.float32),
    )(ai_g, aj_g, edge, ea, g, wae1, wg1a, wg2a, we2, wae2, wg1b, wg2b)


def kernel(x, edge, edge_index, edge_attr, node_attr, additional_message_features,
           edge_dist_gauss, W_m1, Wa_m1, W_m2, Wa_m2, W_u1, Wa_u1, W_u2, Wa_u2,
           W_e1, Wa_e1, Wg1a, Wg2a, W_e2, Wa_e2, Wg1b, Wg2b):
    src = edge_index[0]
    dst = edge_index[1]
    dst2 = dst.reshape(E // _SC_CH, _SC_CH)
    zeros_half = jnp.zeros((_NP, DH), dtype=jnp.float32)

    w4 = W_m1[:4]
    yi, yj = _node_proj(x, W_m1[4:4 + D], W_m1[4 + D:4 + 2 * D])
    yig, yjg = _gather_pair(yi, yj, dst, src)
    m2s = _message(yig, yjg, additional_message_features, edge, edge_attr,
                   w4, W_m1[4 + 2 * D:], Wa_m1, W_m2, Wa_m2)
    agg3 = _segment_sum(m2s, dst2, zeros_half)[:, :N, :]

    x_new, ai, aj = _node_update(x, agg3, node_attr, W_u1[:D], W_u1[D:],
                                 Wa_u1, W_u2, Wa_u2, W_e1[:D], W_e1[D:])

    aig, ajg = _gather_pair(ai, aj, dst, src)
    edge_new = _edge_update(aig, ajg, edge, edge_attr, edge_dist_gauss,
                            Wa_e1, Wg1a, Wg2a, W_e2, Wa_e2, Wg1b, Wg2b)
    return (x_new, edge_new)
